# baseline (speedup 1.0000x reference)
import functools

import jax
import jax.numpy as jnp
import numpy as np
from jax.experimental import pallas as pl
from jax.experimental.pallas import tpu as pltpu

NUM_BLOCKS = 3
INNER = 128
EMB = 256
NUM_CENTER_PTS = [2048, 1024, 512]
NUM_NNS = [16, 32, 64]


def _batchnorm(x, g, b):
    m = jnp.mean(x, axis=0)
    v = jnp.var(x, axis=0)
    return (x - m) / jnp.sqrt(v + 1e-5) * g + b


def _fps_single(pts, K):
    Tn = pts.shape[0]

    def body(i, st):
        idx, d = st
        last = pts[idx[i - 1]]
        dn = jnp.sum((pts - last) ** 2, axis=-1)
        d = jnp.minimum(d, dn)
        nxt = jnp.argmax(d).astype(jnp.int32)
        return idx.at[i].set(nxt), d

    idx0 = jnp.zeros((K,), dtype=jnp.int32)
    d0 = jnp.full((Tn,), jnp.inf, dtype=jnp.float32)
    idx, _ = jax.lax.fori_loop(1, K, body, (idx0, d0))
    return idx


def _gather_dim1(x, idx):
    return jax.vmap(lambda xb, ib: xb[ib])(x, idx)


def _nn_search(q, t, K):
    d = (jnp.sum(q ** 2, -1)[:, :, None] + jnp.sum(t ** 2, -1)[:, None, :]
         - 2.0 * jnp.einsum('bqc,btc->bqt', q, t))
    _, idx = jax.lax.top_k(-d, K)
    return idx


def _final_kernel(x_ref, w_ref, b_ref, g_ref, beta_ref, o_ref):
    x = x_ref[...]
    e = jnp.dot(x, w_ref[...], preferred_element_type=jnp.float32) + b_ref[...]
    n = e.shape[0]
    m = jnp.sum(e, axis=0, keepdims=True) / n
    v = jnp.sum((e - m) ** 2, axis=0, keepdims=True) / n
    e = (e - m) / jnp.sqrt(v + 1e-5) * g_ref[...] + beta_ref[...]
    e = jnp.maximum(e, 0.0)
    B = o_ref.shape[0]
    e = e.reshape(B, n // B, e.shape[-1])
    o_ref[...] = jnp.max(e, axis=1)


def _final_stage(tokens_t, params):
    B, Tt, C = tokens_t.shape
    x = tokens_t.reshape(B * Tt, C)
    out = pl.pallas_call(
        _final_kernel,
        out_shape=jax.ShapeDtypeStruct((B, EMB), jnp.float32),
    )(x, params['emb_w'], params['emb_b'].reshape(1, EMB),
      params['emb_g'].reshape(1, EMB), params['emb_beta'].reshape(1, EMB))
    return out


def kernel(tokens, centers, lrfs, params):
    Bn, Tn, C = tokens.shape
    inner = C
    centers_t = centers
    tokens_t = tokens
    T_t = Tn
    for i in range(NUM_BLOCKS):
        Tq = NUM_CENTER_PTS[i]
        k = NUM_NNS[i]
        rep_idx = jax.vmap(lambda p: _fps_single(p, Tq))(centers_t)
        centers_q = _gather_dim1(centers_t, rep_idx)
        tokens_q = _gather_dim1(tokens_t, rep_idx)
        nn_idx = _nn_search(centers_q, centers_t, k)
        skip = tokens_q.reshape(Bn * Tq, inner)
        if i > 0:
            t = jnp.concatenate([tokens_q, tokens_t], axis=1).reshape(
                Bn * (Tq + T_t), inner)
            t = _batchnorm(t, params['bn1_g%d' % i], params['bn1_b%d' % i])
            t = t.reshape(Bn, Tq + T_t, inner)
            tokens_q = t[:, :Tq, :]
            tokens_t = t[:, Tq:, :]
        Q = tokens_q @ params['Wq%d' % i]
        KV = tokens_t @ params['Wkv%d' % i]
        Kx = KV[..., :inner]
        Vx = KV[..., inner:]
        Kx = _gather_dim1(Kx, nn_idx)
        Vx = _gather_dim1(Vx, nn_idx)
        Qr = Q.reshape(Bn * Tq, 1, inner)
        Kr = Kx.reshape(Bn * Tq, k, inner)
        Vr = Vx.reshape(Bn * Tq, k, inner)
        QK = jnp.einsum('bic,bjc->bij', Qr, Kr) / np.sqrt(inner)
        A = jax.nn.softmax(QK, axis=2)
        out = jnp.einsum('bij,bjc->bic', A, Vr).reshape(Bn * Tq, inner)
        tq = out + skip
        skip2 = tq
        tq = _batchnorm(tq, params['bn2_g%d' % i], params['bn2_b%d' % i])
        tq = jax.nn.relu(tq @ params['mlp_w1_%d' % i] + params['mlp_b1_%d' % i])
        tq = tq @ params['mlp_w2_%d' % i] + params['mlp_b2_%d' % i]
        tq = tq + skip2
        tokens_t = tq.reshape(Bn, Tq, inner)
        centers_t = centers_q
        T_t = Tq
    return _final_stage(tokens_t, params)


# speedup vs baseline: 2.2917x; 2.2917x over previous
import functools

import jax
import jax.numpy as jnp
import numpy as np
from jax.experimental import pallas as pl
from jax.experimental.pallas import tpu as pltpu

NUM_BLOCKS = 3
INNER = 128
EMB = 256
NUM_CENTER_PTS = [2048, 1024, 512]
NUM_NNS = [16, 32, 64]


def _batchnorm(x, g, b):
    m = jnp.mean(x, axis=0)
    v = jnp.var(x, axis=0)
    return (x - m) / jnp.sqrt(v + 1e-5) * g + b


def _fps_single(pts, K):
    Tn = pts.shape[0]

    def body(i, st):
        idx, d = st
        last = pts[idx[i - 1]]
        dn = jnp.sum((pts - last) ** 2, axis=-1)
        d = jnp.minimum(d, dn)
        nxt = jnp.argmax(d).astype(jnp.int32)
        return idx.at[i].set(nxt), d

    idx0 = jnp.zeros((K,), dtype=jnp.int32)
    d0 = jnp.full((Tn,), jnp.inf, dtype=jnp.float32)
    idx, _ = jax.lax.fori_loop(1, K, body, (idx0, d0))
    return idx


def _gather_dim1(x, idx):
    return jax.vmap(lambda xb, ib: xb[ib])(x, idx)


def _nn_search(q, t, K):
    d = (jnp.sum(q ** 2, -1)[:, :, None] + jnp.sum(t ** 2, -1)[:, None, :]
         - 2.0 * jnp.einsum('bqc,btc->bqt', q, t))
    _, idx = jax.lax.top_k(-d, K)
    return idx


def _fps_kernel(K, cx_ref, cy_ref, cz_ref, idx_ref, qx_ref, qy_ref, qz_ref):
    x = cx_ref[...]
    y = cy_ref[...]
    z = cz_ref[...]
    B, S, L = x.shape
    lin = (jax.lax.broadcasted_iota(jnp.int32, (1, S, L), 1) * L
           + jax.lax.broadcasted_iota(jnp.int32, (1, S, L), 2))
    lastx = x[:, 0:1, 0:1]
    lasty = y[:, 0:1, 0:1]
    lastz = z[:, 0:1, 0:1]
    idx_ref[:, 0:1, :] = jnp.zeros((B, 1, 1), jnp.int32)
    qx_ref[:, 0:1, :] = lastx
    qy_ref[:, 0:1, :] = lasty
    qz_ref[:, 0:1, :] = lastz
    d0 = jnp.full((B, S, L), jnp.inf, dtype=jnp.float32)

    def body(i, st):
        d, lx, ly, lz = st
        dx = x - lx
        dy = y - ly
        dz = z - lz
        dn = dx * dx + dy * dy + dz * dz
        d = jnp.minimum(d, dn)
        m = jnp.max(d, axis=(1, 2), keepdims=True)
        cand = jnp.where(d == m, lin, jnp.int32(2147483647))
        nxt = jnp.min(cand, axis=(1, 2), keepdims=True)
        hot = lin == nxt
        lx = jnp.sum(jnp.where(hot, x, 0.0), axis=(1, 2), keepdims=True)
        ly = jnp.sum(jnp.where(hot, y, 0.0), axis=(1, 2), keepdims=True)
        lz = jnp.sum(jnp.where(hot, z, 0.0), axis=(1, 2), keepdims=True)
        idx_ref[:, pl.ds(i, 1), :] = nxt
        qx_ref[:, pl.ds(i, 1), :] = lx
        qy_ref[:, pl.ds(i, 1), :] = ly
        qz_ref[:, pl.ds(i, 1), :] = lz
        return d, lx, ly, lz

    jax.lax.fori_loop(1, K, body, (d0, lastx, lasty, lastz))


def _fps_stage(centers_t, K):
    B, T, _ = centers_t.shape
    S = T // 128
    cx = centers_t[..., 0].reshape(B, S, 128)
    cy = centers_t[..., 1].reshape(B, S, 128)
    cz = centers_t[..., 2].reshape(B, S, 128)
    out_shapes = (
        jax.ShapeDtypeStruct((B, K, 1), jnp.int32),
        jax.ShapeDtypeStruct((B, K, 1), jnp.float32),
        jax.ShapeDtypeStruct((B, K, 1), jnp.float32),
        jax.ShapeDtypeStruct((B, K, 1), jnp.float32),
    )
    idx, qx, qy, qz = pl.pallas_call(
        functools.partial(_fps_kernel, K),
        out_shape=out_shapes,
    )(cx, cy, cz)
    rep_idx = idx[..., 0]
    centers_q = jnp.concatenate([qx, qy, qz], axis=-1)
    return rep_idx, centers_q


def _final_kernel(x_ref, w_ref, b_ref, g_ref, beta_ref, o_ref):
    x = x_ref[...]
    e = jnp.dot(x, w_ref[...], preferred_element_type=jnp.float32) + b_ref[...]
    n = e.shape[0]
    m = jnp.sum(e, axis=0, keepdims=True) / n
    v = jnp.sum((e - m) ** 2, axis=0, keepdims=True) / n
    e = (e - m) / jnp.sqrt(v + 1e-5) * g_ref[...] + beta_ref[...]
    e = jnp.maximum(e, 0.0)
    B = o_ref.shape[0]
    e = e.reshape(B, n // B, e.shape[-1])
    o_ref[...] = jnp.max(e, axis=1)


def _final_stage(tokens_t, params):
    B, Tt, C = tokens_t.shape
    x = tokens_t.reshape(B * Tt, C)
    out = pl.pallas_call(
        _final_kernel,
        out_shape=jax.ShapeDtypeStruct((B, EMB), jnp.float32),
    )(x, params['emb_w'], params['emb_b'].reshape(1, EMB),
      params['emb_g'].reshape(1, EMB), params['emb_beta'].reshape(1, EMB))
    return out


def kernel(tokens, centers, lrfs, params):
    Bn, Tn, C = tokens.shape
    inner = C
    centers_t = centers
    tokens_t = tokens
    T_t = Tn
    for i in range(NUM_BLOCKS):
        Tq = NUM_CENTER_PTS[i]
        k = NUM_NNS[i]
        rep_idx, centers_q = _fps_stage(centers_t, Tq)
        tokens_q = _gather_dim1(tokens_t, rep_idx)
        nn_idx = _nn_search(centers_q, centers_t, k)
        skip = tokens_q.reshape(Bn * Tq, inner)
        if i > 0:
            t = jnp.concatenate([tokens_q, tokens_t], axis=1).reshape(
                Bn * (Tq + T_t), inner)
            t = _batchnorm(t, params['bn1_g%d' % i], params['bn1_b%d' % i])
            t = t.reshape(Bn, Tq + T_t, inner)
            tokens_q = t[:, :Tq, :]
            tokens_t = t[:, Tq:, :]
        Q = tokens_q @ params['Wq%d' % i]
        KV = tokens_t @ params['Wkv%d' % i]
        Kx = KV[..., :inner]
        Vx = KV[..., inner:]
        Kx = _gather_dim1(Kx, nn_idx)
        Vx = _gather_dim1(Vx, nn_idx)
        Qr = Q.reshape(Bn * Tq, 1, inner)
        Kr = Kx.reshape(Bn * Tq, k, inner)
        Vr = Vx.reshape(Bn * Tq, k, inner)
        QK = jnp.einsum('bic,bjc->bij', Qr, Kr) / np.sqrt(inner)
        A = jax.nn.softmax(QK, axis=2)
        out = jnp.einsum('bij,bjc->bic', A, Vr).reshape(Bn * Tq, inner)
        tq = out + skip
        skip2 = tq
        tq = _batchnorm(tq, params['bn2_g%d' % i], params['bn2_b%d' % i])
        tq = jax.nn.relu(tq @ params['mlp_w1_%d' % i] + params['mlp_b1_%d' % i])
        tq = tq @ params['mlp_w2_%d' % i] + params['mlp_b2_%d' % i]
        tq = tq + skip2
        tokens_t = tq.reshape(Bn, Tq, inner)
        centers_t = centers_q
        T_t = Tq
    return _final_stage(tokens_t, params)


# speedup vs baseline: 13.5307x; 5.9041x over previous
import functools

import jax
import jax.numpy as jnp
from jax.experimental import pallas as pl
from jax.experimental.pallas import tpu as pltpu

NUM_BLOCKS = 3
INNER = 128
EMB = 256
NUM_CENTER_PTS = [2048, 1024, 512]
NUM_NNS = [16, 32, 64]
QT = 128


def _gather_dim1(x, idx):
    return jax.vmap(lambda xb, ib: xb[ib])(x, idx)


def _fps_kernel(K, cx_ref, cy_ref, cz_ref, idx_ref, qx_ref, qy_ref, qz_ref):
    x = cx_ref[...]
    y = cy_ref[...]
    z = cz_ref[...]
    B, S, L = x.shape
    lin = (jax.lax.broadcasted_iota(jnp.int32, (1, S, L), 1) * L
           + jax.lax.broadcasted_iota(jnp.int32, (1, S, L), 2))
    lastx = x[:, 0:1, 0:1]
    lasty = y[:, 0:1, 0:1]
    lastz = z[:, 0:1, 0:1]
    idx_ref[:, 0:1, :] = jnp.zeros((B, 1, 1), jnp.int32)
    qx_ref[:, 0:1, :] = lastx
    qy_ref[:, 0:1, :] = lasty
    qz_ref[:, 0:1, :] = lastz
    d0 = jnp.full((B, S, L), jnp.inf, dtype=jnp.float32)

    def body(i, st):
        d, lx, ly, lz = st
        dx = x - lx
        dy = y - ly
        dz = z - lz
        dn = dx * dx + dy * dy + dz * dz
        d = jnp.minimum(d, dn)
        m = jnp.max(d, axis=(1, 2), keepdims=True)
        cand = jnp.where(d == m, lin, jnp.int32(2147483647))
        nxt = jnp.min(cand, axis=(1, 2), keepdims=True)
        hot = lin == nxt
        lx = jnp.sum(jnp.where(hot, x, 0.0), axis=(1, 2), keepdims=True)
        ly = jnp.sum(jnp.where(hot, y, 0.0), axis=(1, 2), keepdims=True)
        lz = jnp.sum(jnp.where(hot, z, 0.0), axis=(1, 2), keepdims=True)
        idx_ref[:, pl.ds(i, 1), :] = nxt
        qx_ref[:, pl.ds(i, 1), :] = lx
        qy_ref[:, pl.ds(i, 1), :] = ly
        qz_ref[:, pl.ds(i, 1), :] = lz
        return d, lx, ly, lz

    jax.lax.fori_loop(1, K, body, (d0, lastx, lasty, lastz))


def _fps_stage(ccoords, K):
    cx, cy, cz = ccoords
    B = cx.shape[0]
    out_shapes = (
        jax.ShapeDtypeStruct((B, K, 1), jnp.int32),
        jax.ShapeDtypeStruct((B, K, 1), jnp.float32),
        jax.ShapeDtypeStruct((B, K, 1), jnp.float32),
        jax.ShapeDtypeStruct((B, K, 1), jnp.float32),
    )
    idx, qx, qy, qz = pl.pallas_call(
        functools.partial(_fps_kernel, K),
        out_shape=out_shapes,
    )(cx, cy, cz)
    return idx[..., 0], (qx, qy, qz)


def _stats_kernel(*refs):
    o_ref = refs[-1]
    s = jnp.zeros((1, INNER), jnp.float32)
    q = jnp.zeros((1, INNER), jnp.float32)
    for r in refs[:-1]:
        x = r[...]
        s = s + jnp.sum(x, axis=0, keepdims=True)
        q = q + jnp.sum(x * x, axis=0, keepdims=True)
    o_ref[...] = jnp.concatenate(
        [s, q, jnp.zeros((6, INNER), jnp.float32)], axis=0)


def _stats_stage(arrays):
    return pl.pallas_call(
        _stats_kernel,
        out_shape=jax.ShapeDtypeStruct((8, INNER), jnp.float32),
    )(*arrays)


def _prep_kernel(nrows, tq_ref, tt_ref, wq_ref, wkv_ref, st_ref, g_ref,
                 b_ref, q_ref, k_ref, v_ref):
    tq = tq_ref[0]
    tt = tt_ref[0]
    if st_ref is not None:
        mean = st_ref[0:1, :] / nrows
        var = st_ref[1:2, :] / nrows - mean * mean
        inv = jax.lax.rsqrt(var + 1e-5) * g_ref[...]
        shift = b_ref[...] - mean * inv
        tq = tq * inv + shift
        tt = tt * inv + shift
    q_ref[0] = jnp.dot(tq, wq_ref[...], preferred_element_type=jnp.float32)
    kv = jnp.dot(tt, wkv_ref[...], preferred_element_type=jnp.float32)
    k_ref[0] = kv[:, :INNER]
    v_ref[0] = kv[:, INNER:]


def _prep_stage(tokens_q, tokens_t, wq, wkv, stats, g, b):
    B, K, _ = tokens_q.shape
    T = tokens_t.shape[1]
    has_bn = stats is not None
    nrows = B * (K + T)
    in_specs = [
        pl.BlockSpec((1, K, INNER), lambda i: (i, 0, 0)),
        pl.BlockSpec((1, T, INNER), lambda i: (i, 0, 0)),
        pl.BlockSpec((INNER, INNER), lambda i: (0, 0)),
        pl.BlockSpec((INNER, 2 * INNER), lambda i: (0, 0)),
    ]
    args = [tokens_q, tokens_t, wq, wkv]
    if has_bn:
        in_specs += [
            pl.BlockSpec((8, INNER), lambda i: (0, 0)),
            pl.BlockSpec((1, INNER), lambda i: (0, 0)),
            pl.BlockSpec((1, INNER), lambda i: (0, 0)),
        ]
        args += [stats, g.reshape(1, INNER), b.reshape(1, INNER)]
        body = functools.partial(_prep_kernel, nrows)
    else:
        body = (lambda tqr, ttr, wqr, wkvr, qr, kr, vr:
                _prep_kernel(nrows, tqr, ttr, wqr, wkvr, None, None, None,
                             qr, kr, vr))
    out_shapes = (
        jax.ShapeDtypeStruct((B, K, INNER), jnp.float32),
        jax.ShapeDtypeStruct((B, T, INNER), jnp.float32),
        jax.ShapeDtypeStruct((B, T, INNER), jnp.float32),
    )
    out_specs = (
        pl.BlockSpec((1, K, INNER), lambda i: (i, 0, 0)),
        pl.BlockSpec((1, T, INNER), lambda i: (i, 0, 0)),
        pl.BlockSpec((1, T, INNER), lambda i: (i, 0, 0)),
    )
    return pl.pallas_call(
        body, grid=(B,), in_specs=in_specs, out_specs=out_specs,
        out_shape=out_shapes,
    )(*args)


def _attn_kernel(k, scale, qx_ref, qy_ref, qz_ref, tx_ref, ty_ref, tz_ref,
                 q_ref, kx_ref, vx_ref, skip_ref, o_ref, dw_ref, nm_ref):
    qx = qx_ref[0]
    qy = qy_ref[0]
    qz = qz_ref[0]
    tx = tx_ref[0]
    ty = ty_ref[0]
    tz = tz_ref[0]
    T = tx.shape[1]
    n_q = qx.shape[0]
    sq = qx * qx + qy * qy + qz * qz
    st = tx * tx + ty * ty + tz * tz
    qt = qx * tx + qy * ty + qz * tz
    d = (sq + st) - 2.0 * qt
    lin = jax.lax.broadcasted_iota(jnp.int32, (n_q, T), 1)
    dw_ref[...] = d
    nm_ref[...] = jnp.full((n_q, T), -1e30, jnp.float32)

    def body(_, carry):
        dwork = dw_ref[...]
        m = jnp.min(dwork, axis=1, keepdims=True)
        cand = jnp.where(dwork == m, lin, jnp.int32(2147483647))
        ix = jnp.min(cand, axis=1, keepdims=True)
        hot = lin == ix
        dw_ref[...] = jnp.where(hot, jnp.inf, dwork)
        nm_ref[...] = jnp.where(hot, 0.0, nm_ref[...])
        return carry

    jax.lax.fori_loop(0, k, body, 0)
    qk = jax.lax.dot_general(
        q_ref[0], kx_ref[0], (((1,), (1,)), ((), ())),
        preferred_element_type=jnp.float32)
    logits = qk * scale + nm_ref[...]
    mx = jnp.max(logits, axis=1, keepdims=True)
    e = jnp.exp(logits - mx)
    a = e / jnp.sum(e, axis=1, keepdims=True)
    av = jnp.dot(a, vx_ref[0], preferred_element_type=jnp.float32)
    o_ref[0] = av + skip_ref[0]


def _attn_stage(qcoords, tcoords_lane, Q, Kx, Vx, skip, k):
    B, K, _ = Q.shape
    T = Kx.shape[1]
    scale = 1.0 / (float(INNER) ** 0.5)
    qspec = pl.BlockSpec((1, QT, 1), lambda b, t: (b, t, 0))
    tspec = pl.BlockSpec((1, 1, T), lambda b, t: (b, 0, 0))
    in_specs = [qspec, qspec, qspec, tspec, tspec, tspec,
                pl.BlockSpec((1, QT, INNER), lambda b, t: (b, t, 0)),
                pl.BlockSpec((1, T, INNER), lambda b, t: (b, 0, 0)),
                pl.BlockSpec((1, T, INNER), lambda b, t: (b, 0, 0)),
                pl.BlockSpec((1, QT, INNER), lambda b, t: (b, t, 0))]
    return pl.pallas_call(
        functools.partial(_attn_kernel, k, scale),
        grid=(B, K // QT),
        in_specs=in_specs,
        out_specs=pl.BlockSpec((1, QT, INNER), lambda b, t: (b, t, 0)),
        out_shape=jax.ShapeDtypeStruct((B, K, INNER), jnp.float32),
        scratch_shapes=[pltpu.VMEM((QT, T), jnp.float32),
                        pltpu.VMEM((QT, T), jnp.float32)],
    )(*qcoords, *tcoords_lane, Q, Kx, Vx, skip)


def _tail_kernel(nrows, x_ref, st_ref, w1_ref, b1_ref, w2_ref, b2_ref,
                 g_ref, b_ref, o_ref):
    x = x_ref[...]
    mean = st_ref[0:1, :] / nrows
    var = st_ref[1:2, :] / nrows - mean * mean
    inv = jax.lax.rsqrt(var + 1e-5) * g_ref[...]
    shift = b_ref[...] - mean * inv
    xn = x * inv + shift
    h = jnp.maximum(
        jnp.dot(xn, w1_ref[...], preferred_element_type=jnp.float32)
        + b1_ref[...], 0.0)
    y = (jnp.dot(h, w2_ref[...], preferred_element_type=jnp.float32)
         + b2_ref[...])
    o_ref[...] = y + x


def _tail_stage(x, stats, w1, b1, w2, b2, g, b):
    N = x.shape[0]
    R = 1024
    in_specs = [
        pl.BlockSpec((R, INNER), lambda i: (i, 0)),
        pl.BlockSpec((8, INNER), lambda i: (0, 0)),
        pl.BlockSpec((INNER, 2 * INNER), lambda i: (0, 0)),
        pl.BlockSpec((1, 2 * INNER), lambda i: (0, 0)),
        pl.BlockSpec((2 * INNER, INNER), lambda i: (0, 0)),
        pl.BlockSpec((1, INNER), lambda i: (0, 0)),
        pl.BlockSpec((1, INNER), lambda i: (0, 0)),
        pl.BlockSpec((1, INNER), lambda i: (0, 0)),
    ]
    return pl.pallas_call(
        functools.partial(_tail_kernel, N),
        grid=(N // R,),
        in_specs=in_specs,
        out_specs=pl.BlockSpec((R, INNER), lambda i: (i, 0)),
        out_shape=jax.ShapeDtypeStruct((N, INNER), jnp.float32),
    )(x, stats, w1, b1.reshape(1, 2 * INNER), w2, b2.reshape(1, INNER),
      g.reshape(1, INNER), b.reshape(1, INNER))


def _final_kernel(x_ref, w_ref, b_ref, g_ref, beta_ref, o_ref):
    x = x_ref[...]
    e = jnp.dot(x, w_ref[...], preferred_element_type=jnp.float32) + b_ref[...]
    n = e.shape[0]
    m = jnp.sum(e, axis=0, keepdims=True) / n
    v = jnp.sum((e - m) ** 2, axis=0, keepdims=True) / n
    e = (e - m) / jnp.sqrt(v + 1e-5) * g_ref[...] + beta_ref[...]
    e = jnp.maximum(e, 0.0)
    B = o_ref.shape[0]
    e = e.reshape(B, n // B, e.shape[-1])
    o_ref[...] = jnp.max(e, axis=1)


def _final_stage(tokens_t, params):
    B, Tt, C = tokens_t.shape
    x = tokens_t.reshape(B * Tt, C)
    return pl.pallas_call(
        _final_kernel,
        out_shape=jax.ShapeDtypeStruct((B, EMB), jnp.float32),
    )(x, params['emb_w'], params['emb_b'].reshape(1, EMB),
      params['emb_g'].reshape(1, EMB), params['emb_beta'].reshape(1, EMB))


def kernel(tokens, centers, lrfs, params):
    B, T, C = tokens.shape
    tokens_t = tokens
    ccoords = tuple(
        centers[..., c].reshape(B, T // 128, 128) for c in range(3))
    T_t = T
    for i in range(NUM_BLOCKS):
        Tq = NUM_CENTER_PTS[i]
        k = NUM_NNS[i]
        rep_idx, qcoords = _fps_stage(ccoords, Tq)
        tokens_q = _gather_dim1(tokens_t, rep_idx)
        if i > 0:
            stats1 = _stats_stage([tokens_q.reshape(B * Tq, C),
                                   tokens_t.reshape(B * T_t, C)])
            g1, b1 = params['bn1_g%d' % i], params['bn1_b%d' % i]
        else:
            stats1, g1, b1 = None, None, None
        Q, Kx, Vx = _prep_stage(tokens_q, tokens_t,
                                params['Wq%d' % i], params['Wkv%d' % i],
                                stats1, g1, b1)
        tcoords_lane = tuple(cc.reshape(B, 1, T_t) for cc in ccoords)
        attn = _attn_stage(qcoords, tcoords_lane, Q, Kx, Vx, tokens_q, k)
        x = attn.reshape(B * Tq, C)
        stats2 = _stats_stage([x])
        y = _tail_stage(x, stats2,
                        params['mlp_w1_%d' % i], params['mlp_b1_%d' % i],
                        params['mlp_w2_%d' % i], params['mlp_b2_%d' % i],
                        params['bn2_g%d' % i], params['bn2_b%d' % i])
        tokens_t = y.reshape(B, Tq, C)
        ccoords = tuple(qc.reshape(B, Tq // 128, 128) for qc in qcoords)
        T_t = Tq
    return _final_stage(tokens_t, params)


# speedup vs baseline: 14.9297x; 1.1034x over previous
import functools

import jax
import jax.numpy as jnp
from jax.experimental import pallas as pl
from jax.experimental.pallas import tpu as pltpu

NUM_BLOCKS = 3
INNER = 128
EMB = 256
NUM_CENTER_PTS = [2048, 1024, 512]
NUM_NNS = [16, 32, 64]
QT = 128


def _gather_dim1(x, idx):
    return jax.vmap(lambda xb, ib: xb[ib])(x, idx)


def _fps_kernel(K, cx_ref, cy_ref, cz_ref, idx_ref, qx_ref, qy_ref, qz_ref):
    x = cx_ref[...]
    y = cy_ref[...]
    z = cz_ref[...]
    B, S, L = x.shape
    lin = (jax.lax.broadcasted_iota(jnp.int32, (1, S, L), 1) * L
           + jax.lax.broadcasted_iota(jnp.int32, (1, S, L), 2))
    lastx = x[:, 0:1, 0:1]
    lasty = y[:, 0:1, 0:1]
    lastz = z[:, 0:1, 0:1]
    idx_ref[:, 0:1, :] = jnp.zeros((B, 1, 1), jnp.int32)
    qx_ref[:, 0:1, :] = lastx
    qy_ref[:, 0:1, :] = lasty
    qz_ref[:, 0:1, :] = lastz
    d0 = jnp.full((B, S, L), jnp.inf, dtype=jnp.float32)

    def body(i, st):
        d, lx, ly, lz = st
        dx = x - lx
        dy = y - ly
        dz = z - lz
        dn = dx * dx + dy * dy + dz * dz
        d = jnp.minimum(d, dn)
        m = jnp.max(d, axis=(1, 2), keepdims=True)
        cand = jnp.where(d == m, lin, jnp.int32(2147483647))
        nxt = jnp.min(cand, axis=(1, 2), keepdims=True)
        hot = lin == nxt
        lx = jnp.sum(jnp.where(hot, x, 0.0), axis=(1, 2), keepdims=True)
        ly = jnp.sum(jnp.where(hot, y, 0.0), axis=(1, 2), keepdims=True)
        lz = jnp.sum(jnp.where(hot, z, 0.0), axis=(1, 2), keepdims=True)
        idx_ref[:, pl.ds(i, 1), :] = nxt
        qx_ref[:, pl.ds(i, 1), :] = lx
        qy_ref[:, pl.ds(i, 1), :] = ly
        qz_ref[:, pl.ds(i, 1), :] = lz
        return d, lx, ly, lz

    jax.lax.fori_loop(1, K, body, (d0, lastx, lasty, lastz))


def _fps_stage(ccoords, K):
    cx, cy, cz = ccoords
    B = cx.shape[0]
    out_shapes = (
        jax.ShapeDtypeStruct((B, K, 1), jnp.int32),
        jax.ShapeDtypeStruct((B, K, 1), jnp.float32),
        jax.ShapeDtypeStruct((B, K, 1), jnp.float32),
        jax.ShapeDtypeStruct((B, K, 1), jnp.float32),
    )
    idx, qx, qy, qz = pl.pallas_call(
        functools.partial(_fps_kernel, K),
        out_shape=out_shapes,
    )(cx, cy, cz)
    return idx[..., 0], (qx, qy, qz)


def _stats_kernel(*refs):
    o_ref = refs[-1]
    s = jnp.zeros((1, INNER), jnp.float32)
    q = jnp.zeros((1, INNER), jnp.float32)
    for r in refs[:-1]:
        x = r[...]
        s = s + jnp.sum(x, axis=0, keepdims=True)
        q = q + jnp.sum(x * x, axis=0, keepdims=True)
    o_ref[...] = jnp.concatenate(
        [s, q, jnp.zeros((6, INNER), jnp.float32)], axis=0)


def _stats_stage(arrays):
    return pl.pallas_call(
        _stats_kernel,
        out_shape=jax.ShapeDtypeStruct((8, INNER), jnp.float32),
    )(*arrays)


def _prep_kernel(nrows, tq_ref, tt_ref, wq_ref, wkv_ref, st_ref, g_ref,
                 b_ref, q_ref, k_ref, v_ref):
    tq = tq_ref[0]
    tt = tt_ref[0]
    if st_ref is not None:
        mean = st_ref[0:1, :] / nrows
        var = st_ref[1:2, :] / nrows - mean * mean
        inv = jax.lax.rsqrt(var + 1e-5) * g_ref[...]
        shift = b_ref[...] - mean * inv
        tq = tq * inv + shift
        tt = tt * inv + shift
    q_ref[0] = jnp.dot(tq, wq_ref[...], preferred_element_type=jnp.float32)
    kv = jnp.dot(tt, wkv_ref[...], preferred_element_type=jnp.float32)
    k_ref[0] = kv[:, :INNER]
    v_ref[0] = kv[:, INNER:]


def _prep_stage(tokens_q, tokens_t, wq, wkv, stats, g, b):
    B, K, _ = tokens_q.shape
    T = tokens_t.shape[1]
    has_bn = stats is not None
    nrows = B * (K + T)
    in_specs = [
        pl.BlockSpec((1, K, INNER), lambda i: (i, 0, 0)),
        pl.BlockSpec((1, T, INNER), lambda i: (i, 0, 0)),
        pl.BlockSpec((INNER, INNER), lambda i: (0, 0)),
        pl.BlockSpec((INNER, 2 * INNER), lambda i: (0, 0)),
    ]
    args = [tokens_q, tokens_t, wq, wkv]
    if has_bn:
        in_specs += [
            pl.BlockSpec((8, INNER), lambda i: (0, 0)),
            pl.BlockSpec((1, INNER), lambda i: (0, 0)),
            pl.BlockSpec((1, INNER), lambda i: (0, 0)),
        ]
        args += [stats, g.reshape(1, INNER), b.reshape(1, INNER)]
        body = functools.partial(_prep_kernel, nrows)
    else:
        body = (lambda tqr, ttr, wqr, wkvr, qr, kr, vr:
                _prep_kernel(nrows, tqr, ttr, wqr, wkvr, None, None, None,
                             qr, kr, vr))
    out_shapes = (
        jax.ShapeDtypeStruct((B, K, INNER), jnp.float32),
        jax.ShapeDtypeStruct((B, T, INNER), jnp.float32),
        jax.ShapeDtypeStruct((B, T, INNER), jnp.float32),
    )
    out_specs = (
        pl.BlockSpec((1, K, INNER), lambda i: (i, 0, 0)),
        pl.BlockSpec((1, T, INNER), lambda i: (i, 0, 0)),
        pl.BlockSpec((1, T, INNER), lambda i: (i, 0, 0)),
    )
    return pl.pallas_call(
        body, grid=(B,), in_specs=in_specs, out_specs=out_specs,
        out_shape=out_shapes,
    )(*args)


def _attn_kernel(k, scale, qx_ref, qy_ref, qz_ref, tx_ref, ty_ref, tz_ref,
                 q_ref, kx_ref, vx_ref, skip_ref, o_ref, dw_ref, nm_ref):
    qx = qx_ref[0]
    qy = qy_ref[0]
    qz = qz_ref[0]
    tx = tx_ref[0]
    ty = ty_ref[0]
    tz = tz_ref[0]
    T = tx.shape[1]
    n_q = qx.shape[0]
    sq = qx * qx + qy * qy + qz * qz
    st = tx * tx + ty * ty + tz * tz
    qt = qx * tx + qy * ty + qz * tz
    d = (sq + st) - 2.0 * qt
    bits = jax.lax.bitcast_convert_type(d, jnp.int32)
    dw_ref[...] = jnp.where(bits < 0, bits ^ jnp.int32(0x7FFFFFFF), bits)

    def rbody(i, X):
        shift = jax.lax.shift_left(jnp.int32(1), jnp.int32(31) - i)
        c = X + shift
        cnt = jnp.sum((dw_ref[...] < c).astype(jnp.int32),
                      axis=1, keepdims=True)
        return jnp.where(cnt < k, c, X)

    X = jax.lax.fori_loop(
        0, 32, rbody, jnp.full((n_q, 1), jnp.int32(-2147483648)))
    skey = dw_ref[...]
    less = skey < X
    cnt_less = jnp.sum(less.astype(jnp.int32), axis=1, keepdims=True)
    eq = (skey == X).astype(jnp.int32)
    s = eq
    sh = 1
    while sh < T:
        s = s + jnp.concatenate(
            [jnp.zeros((n_q, sh), jnp.int32), s[:, :T - sh]], axis=1)
        sh *= 2
    sel = less | ((eq == 1) & (s <= (k - cnt_less)))
    nm_ref[...] = jnp.where(sel, 0.0, -1e30)
    qk = jax.lax.dot_general(
        q_ref[0], kx_ref[0], (((1,), (1,)), ((), ())),
        preferred_element_type=jnp.float32)
    logits = qk * scale + nm_ref[...]
    mx = jnp.max(logits, axis=1, keepdims=True)
    e = jnp.exp(logits - mx)
    a = e / jnp.sum(e, axis=1, keepdims=True)
    av = jnp.dot(a, vx_ref[0], preferred_element_type=jnp.float32)
    o_ref[0] = av + skip_ref[0]


def _attn_stage(qcoords, tcoords_lane, Q, Kx, Vx, skip, k):
    B, K, _ = Q.shape
    T = Kx.shape[1]
    scale = 1.0 / (float(INNER) ** 0.5)
    qspec = pl.BlockSpec((1, QT, 1), lambda b, t: (b, t, 0))
    tspec = pl.BlockSpec((1, 1, T), lambda b, t: (b, 0, 0))
    in_specs = [qspec, qspec, qspec, tspec, tspec, tspec,
                pl.BlockSpec((1, QT, INNER), lambda b, t: (b, t, 0)),
                pl.BlockSpec((1, T, INNER), lambda b, t: (b, 0, 0)),
                pl.BlockSpec((1, T, INNER), lambda b, t: (b, 0, 0)),
                pl.BlockSpec((1, QT, INNER), lambda b, t: (b, t, 0))]
    return pl.pallas_call(
        functools.partial(_attn_kernel, k, scale),
        grid=(B, K // QT),
        in_specs=in_specs,
        out_specs=pl.BlockSpec((1, QT, INNER), lambda b, t: (b, t, 0)),
        out_shape=jax.ShapeDtypeStruct((B, K, INNER), jnp.float32),
        scratch_shapes=[pltpu.VMEM((QT, T), jnp.int32),
                        pltpu.VMEM((QT, T), jnp.float32)],
    )(*qcoords, *tcoords_lane, Q, Kx, Vx, skip)


def _tail_kernel(nrows, x_ref, st_ref, w1_ref, b1_ref, w2_ref, b2_ref,
                 g_ref, b_ref, o_ref):
    x = x_ref[...]
    mean = st_ref[0:1, :] / nrows
    var = st_ref[1:2, :] / nrows - mean * mean
    inv = jax.lax.rsqrt(var + 1e-5) * g_ref[...]
    shift = b_ref[...] - mean * inv
    xn = x * inv + shift
    h = jnp.maximum(
        jnp.dot(xn, w1_ref[...], preferred_element_type=jnp.float32)
        + b1_ref[...], 0.0)
    y = (jnp.dot(h, w2_ref[...], preferred_element_type=jnp.float32)
         + b2_ref[...])
    o_ref[...] = y + x


def _tail_stage(x, stats, w1, b1, w2, b2, g, b):
    N = x.shape[0]
    R = 1024
    in_specs = [
        pl.BlockSpec((R, INNER), lambda i: (i, 0)),
        pl.BlockSpec((8, INNER), lambda i: (0, 0)),
        pl.BlockSpec((INNER, 2 * INNER), lambda i: (0, 0)),
        pl.BlockSpec((1, 2 * INNER), lambda i: (0, 0)),
        pl.BlockSpec((2 * INNER, INNER), lambda i: (0, 0)),
        pl.BlockSpec((1, INNER), lambda i: (0, 0)),
        pl.BlockSpec((1, INNER), lambda i: (0, 0)),
        pl.BlockSpec((1, INNER), lambda i: (0, 0)),
    ]
    return pl.pallas_call(
        functools.partial(_tail_kernel, N),
        grid=(N // R,),
        in_specs=in_specs,
        out_specs=pl.BlockSpec((R, INNER), lambda i: (i, 0)),
        out_shape=jax.ShapeDtypeStruct((N, INNER), jnp.float32),
    )(x, stats, w1, b1.reshape(1, 2 * INNER), w2, b2.reshape(1, INNER),
      g.reshape(1, INNER), b.reshape(1, INNER))


def _final_kernel(x_ref, w_ref, b_ref, g_ref, beta_ref, o_ref):
    x = x_ref[...]
    e = jnp.dot(x, w_ref[...], preferred_element_type=jnp.float32) + b_ref[...]
    n = e.shape[0]
    m = jnp.sum(e, axis=0, keepdims=True) / n
    v = jnp.sum((e - m) ** 2, axis=0, keepdims=True) / n
    e = (e - m) / jnp.sqrt(v + 1e-5) * g_ref[...] + beta_ref[...]
    e = jnp.maximum(e, 0.0)
    B = o_ref.shape[0]
    e = e.reshape(B, n // B, e.shape[-1])
    o_ref[...] = jnp.max(e, axis=1)


def _final_stage(tokens_t, params):
    B, Tt, C = tokens_t.shape
    x = tokens_t.reshape(B * Tt, C)
    return pl.pallas_call(
        _final_kernel,
        out_shape=jax.ShapeDtypeStruct((B, EMB), jnp.float32),
    )(x, params['emb_w'], params['emb_b'].reshape(1, EMB),
      params['emb_g'].reshape(1, EMB), params['emb_beta'].reshape(1, EMB))


def kernel(tokens, centers, lrfs, params):
    B, T, C = tokens.shape
    tokens_t = tokens
    ccoords = tuple(
        centers[..., c].reshape(B, T // 128, 128) for c in range(3))
    T_t = T
    for i in range(NUM_BLOCKS):
        Tq = NUM_CENTER_PTS[i]
        k = NUM_NNS[i]
        rep_idx, qcoords = _fps_stage(ccoords, Tq)
        tokens_q = _gather_dim1(tokens_t, rep_idx)
        if i > 0:
            stats1 = _stats_stage([tokens_q.reshape(B * Tq, C),
                                   tokens_t.reshape(B * T_t, C)])
            g1, b1 = params['bn1_g%d' % i], params['bn1_b%d' % i]
        else:
            stats1, g1, b1 = None, None, None
        Q, Kx, Vx = _prep_stage(tokens_q, tokens_t,
                                params['Wq%d' % i], params['Wkv%d' % i],
                                stats1, g1, b1)
        tcoords_lane = tuple(cc.reshape(B, 1, T_t) for cc in ccoords)
        attn = _attn_stage(qcoords, tcoords_lane, Q, Kx, Vx, tokens_q, k)
        x = attn.reshape(B * Tq, C)
        stats2 = _stats_stage([x])
        y = _tail_stage(x, stats2,
                        params['mlp_w1_%d' % i], params['mlp_b1_%d' % i],
                        params['mlp_w2_%d' % i], params['mlp_b2_%d' % i],
                        params['bn2_g%d' % i], params['bn2_b%d' % i])
        tokens_t = y.reshape(B, Tq, C)
        ccoords = tuple(qc.reshape(B, Tq // 128, 128) for qc in qcoords)
        T_t = Tq
    return _final_stage(tokens_t, params)


# speedup vs baseline: 15.8853x; 1.0640x over previous
import functools

import jax
import jax.numpy as jnp
from jax.experimental import pallas as pl
from jax.experimental.pallas import tpu as pltpu

NUM_BLOCKS = 3
INNER = 128
EMB = 256
NUM_CENTER_PTS = [2048, 1024, 512]
NUM_NNS = [16, 32, 64]
QT = 256


def _gather_dim1(x, idx):
    return jax.vmap(lambda xb, ib: xb[ib])(x, idx)


def _fps_kernel(K, cx_ref, cy_ref, cz_ref, idx_ref, qx_ref, qy_ref, qz_ref):
    x = cx_ref[...]
    y = cy_ref[...]
    z = cz_ref[...]
    B, S, L = x.shape
    lin = (jax.lax.broadcasted_iota(jnp.int32, (1, S, L), 1) * L
           + jax.lax.broadcasted_iota(jnp.int32, (1, S, L), 2))
    lastx = x[:, 0:1, 0:1]
    lasty = y[:, 0:1, 0:1]
    lastz = z[:, 0:1, 0:1]
    idx_ref[:, 0:1, :] = jnp.zeros((B, 1, 1), jnp.int32)
    qx_ref[:, 0:1, :] = lastx
    qy_ref[:, 0:1, :] = lasty
    qz_ref[:, 0:1, :] = lastz
    d0 = jnp.full((B, S, L), jnp.inf, dtype=jnp.float32)

    def body(i, st):
        d, lx, ly, lz = st
        dx = x - lx
        dy = y - ly
        dz = z - lz
        dn = dx * dx + dy * dy + dz * dz
        d = jnp.minimum(d, dn)
        m = jnp.max(d, axis=(1, 2), keepdims=True)
        cand = jnp.where(d == m, lin, jnp.int32(2147483647))
        nxt = jnp.min(cand, axis=(1, 2), keepdims=True)
        hot = lin == nxt
        lx = jnp.sum(jnp.where(hot, x, 0.0), axis=(1, 2), keepdims=True)
        ly = jnp.sum(jnp.where(hot, y, 0.0), axis=(1, 2), keepdims=True)
        lz = jnp.sum(jnp.where(hot, z, 0.0), axis=(1, 2), keepdims=True)
        idx_ref[:, pl.ds(i, 1), :] = nxt
        qx_ref[:, pl.ds(i, 1), :] = lx
        qy_ref[:, pl.ds(i, 1), :] = ly
        qz_ref[:, pl.ds(i, 1), :] = lz
        return d, lx, ly, lz

    jax.lax.fori_loop(1, K, body, (d0, lastx, lasty, lastz))


def _fps_stage(ccoords, K):
    cx, cy, cz = ccoords
    B = cx.shape[0]
    out_shapes = (
        jax.ShapeDtypeStruct((B, K, 1), jnp.int32),
        jax.ShapeDtypeStruct((B, K, 1), jnp.float32),
        jax.ShapeDtypeStruct((B, K, 1), jnp.float32),
        jax.ShapeDtypeStruct((B, K, 1), jnp.float32),
    )
    idx, qx, qy, qz = pl.pallas_call(
        functools.partial(_fps_kernel, K),
        out_shape=out_shapes,
    )(cx, cy, cz)
    return idx[..., 0], (qx, qy, qz)


def _stats_kernel(*refs):
    o_ref = refs[-1]
    s = jnp.zeros((1, INNER), jnp.float32)
    q = jnp.zeros((1, INNER), jnp.float32)
    for r in refs[:-1]:
        x = r[...]
        s = s + jnp.sum(x, axis=0, keepdims=True)
        q = q + jnp.sum(x * x, axis=0, keepdims=True)
    o_ref[...] = jnp.concatenate(
        [s, q, jnp.zeros((6, INNER), jnp.float32)], axis=0)


def _stats_stage(arrays):
    return pl.pallas_call(
        _stats_kernel,
        out_shape=jax.ShapeDtypeStruct((8, INNER), jnp.float32),
    )(*arrays)


def _prep_kernel(nrows, tq_ref, tt_ref, wq_ref, wkv_ref, st_ref, g_ref,
                 b_ref, q_ref, k_ref, v_ref):
    tq = tq_ref[0]
    tt = tt_ref[0]
    if st_ref is not None:
        mean = st_ref[0:1, :] / nrows
        var = st_ref[1:2, :] / nrows - mean * mean
        inv = jax.lax.rsqrt(var + 1e-5) * g_ref[...]
        shift = b_ref[...] - mean * inv
        tq = tq * inv + shift
        tt = tt * inv + shift
    q_ref[0] = jnp.dot(tq, wq_ref[...], preferred_element_type=jnp.float32)
    kv = jnp.dot(tt, wkv_ref[...], preferred_element_type=jnp.float32)
    k_ref[0] = kv[:, :INNER]
    v_ref[0] = kv[:, INNER:]


def _prep_stage(tokens_q, tokens_t, wq, wkv, stats, g, b):
    B, K, _ = tokens_q.shape
    T = tokens_t.shape[1]
    has_bn = stats is not None
    nrows = B * (K + T)
    in_specs = [
        pl.BlockSpec((1, K, INNER), lambda i: (i, 0, 0)),
        pl.BlockSpec((1, T, INNER), lambda i: (i, 0, 0)),
        pl.BlockSpec((INNER, INNER), lambda i: (0, 0)),
        pl.BlockSpec((INNER, 2 * INNER), lambda i: (0, 0)),
    ]
    args = [tokens_q, tokens_t, wq, wkv]
    if has_bn:
        in_specs += [
            pl.BlockSpec((8, INNER), lambda i: (0, 0)),
            pl.BlockSpec((1, INNER), lambda i: (0, 0)),
            pl.BlockSpec((1, INNER), lambda i: (0, 0)),
        ]
        args += [stats, g.reshape(1, INNER), b.reshape(1, INNER)]
        body = functools.partial(_prep_kernel, nrows)
    else:
        body = (lambda tqr, ttr, wqr, wkvr, qr, kr, vr:
                _prep_kernel(nrows, tqr, ttr, wqr, wkvr, None, None, None,
                             qr, kr, vr))
    out_shapes = (
        jax.ShapeDtypeStruct((B, K, INNER), jnp.float32),
        jax.ShapeDtypeStruct((B, T, INNER), jnp.float32),
        jax.ShapeDtypeStruct((B, T, INNER), jnp.float32),
    )
    out_specs = (
        pl.BlockSpec((1, K, INNER), lambda i: (i, 0, 0)),
        pl.BlockSpec((1, T, INNER), lambda i: (i, 0, 0)),
        pl.BlockSpec((1, T, INNER), lambda i: (i, 0, 0)),
    )
    return pl.pallas_call(
        body, grid=(B,), in_specs=in_specs, out_specs=out_specs,
        out_shape=out_shapes,
    )(*args)


def _attn_kernel(k, scale, qx_ref, qy_ref, qz_ref, tx_ref, ty_ref, tz_ref,
                 q_ref, kx_ref, vx_ref, skip_ref, o_ref, dw_ref, nm_ref):
    qx = qx_ref[0]
    qy = qy_ref[0]
    qz = qz_ref[0]
    tx = tx_ref[0]
    ty = ty_ref[0]
    tz = tz_ref[0]
    T = tx.shape[1]
    n_q = qx.shape[0]
    sq = qx * qx + qy * qy + qz * qz
    st = tx * tx + ty * ty + tz * tz
    qt = qx * tx + qy * ty + qz * tz
    d = (sq + st) - 2.0 * qt
    bits = jax.lax.bitcast_convert_type(d, jnp.int32)
    dw_ref[...] = jnp.where(bits < 0, bits ^ jnp.int32(0x7FFFFFFF), bits)

    def rbody(i, X):
        shift = jax.lax.shift_left(jnp.int32(1), jnp.int32(31) - i)
        c = X + shift
        cnt = jnp.sum((dw_ref[...] < c).astype(jnp.int32),
                      axis=1, keepdims=True)
        return jnp.where(cnt < k, c, X)

    X = jax.lax.fori_loop(
        0, 32, rbody, jnp.full((n_q, 1), jnp.int32(-2147483648)))
    skey = dw_ref[...]
    less = skey < X
    cnt_less = jnp.sum(less.astype(jnp.int32), axis=1, keepdims=True)
    eq = (skey == X).astype(jnp.int32)
    s = eq
    sh = 1
    while sh < T:
        s = s + jnp.concatenate(
            [jnp.zeros((n_q, sh), jnp.int32), s[:, :T - sh]], axis=1)
        sh *= 2
    sel = less | ((eq == 1) & (s <= (k - cnt_less)))
    nm_ref[...] = jnp.where(sel, 0.0, -1e30)
    qk = jax.lax.dot_general(
        q_ref[0], kx_ref[0], (((1,), (1,)), ((), ())),
        preferred_element_type=jnp.float32)
    logits = qk * scale + nm_ref[...]
    mx = jnp.max(logits, axis=1, keepdims=True)
    e = jnp.exp(logits - mx)
    a = e / jnp.sum(e, axis=1, keepdims=True)
    av = jnp.dot(a, vx_ref[0], preferred_element_type=jnp.float32)
    o_ref[0] = av + skip_ref[0]


def _attn_stage(qcoords, tcoords_lane, Q, Kx, Vx, skip, k):
    B, K, _ = Q.shape
    T = Kx.shape[1]
    scale = 1.0 / (float(INNER) ** 0.5)
    qspec = pl.BlockSpec((1, QT, 1), lambda b, t: (b, t, 0))
    tspec = pl.BlockSpec((1, 1, T), lambda b, t: (b, 0, 0))
    in_specs = [qspec, qspec, qspec, tspec, tspec, tspec,
                pl.BlockSpec((1, QT, INNER), lambda b, t: (b, t, 0)),
                pl.BlockSpec((1, T, INNER), lambda b, t: (b, 0, 0)),
                pl.BlockSpec((1, T, INNER), lambda b, t: (b, 0, 0)),
                pl.BlockSpec((1, QT, INNER), lambda b, t: (b, t, 0))]
    return pl.pallas_call(
        functools.partial(_attn_kernel, k, scale),
        grid=(B, K // QT),
        in_specs=in_specs,
        out_specs=pl.BlockSpec((1, QT, INNER), lambda b, t: (b, t, 0)),
        out_shape=jax.ShapeDtypeStruct((B, K, INNER), jnp.float32),
        scratch_shapes=[pltpu.VMEM((QT, T), jnp.int32),
                        pltpu.VMEM((QT, T), jnp.float32)],
    )(*qcoords, *tcoords_lane, Q, Kx, Vx, skip)


def _tail_kernel(nrows, x_ref, st_ref, w1_ref, b1_ref, w2_ref, b2_ref,
                 g_ref, b_ref, o_ref):
    x = x_ref[...]
    mean = st_ref[0:1, :] / nrows
    var = st_ref[1:2, :] / nrows - mean * mean
    inv = jax.lax.rsqrt(var + 1e-5) * g_ref[...]
    shift = b_ref[...] - mean * inv
    xn = x * inv + shift
    h = jnp.maximum(
        jnp.dot(xn, w1_ref[...], preferred_element_type=jnp.float32)
        + b1_ref[...], 0.0)
    y = (jnp.dot(h, w2_ref[...], preferred_element_type=jnp.float32)
         + b2_ref[...])
    o_ref[...] = y + x


def _tail_stage(x, stats, w1, b1, w2, b2, g, b):
    N = x.shape[0]
    R = 1024
    in_specs = [
        pl.BlockSpec((R, INNER), lambda i: (i, 0)),
        pl.BlockSpec((8, INNER), lambda i: (0, 0)),
        pl.BlockSpec((INNER, 2 * INNER), lambda i: (0, 0)),
        pl.BlockSpec((1, 2 * INNER), lambda i: (0, 0)),
        pl.BlockSpec((2 * INNER, INNER), lambda i: (0, 0)),
        pl.BlockSpec((1, INNER), lambda i: (0, 0)),
        pl.BlockSpec((1, INNER), lambda i: (0, 0)),
        pl.BlockSpec((1, INNER), lambda i: (0, 0)),
    ]
    return pl.pallas_call(
        functools.partial(_tail_kernel, N),
        grid=(N // R,),
        in_specs=in_specs,
        out_specs=pl.BlockSpec((R, INNER), lambda i: (i, 0)),
        out_shape=jax.ShapeDtypeStruct((N, INNER), jnp.float32),
    )(x, stats, w1, b1.reshape(1, 2 * INNER), w2, b2.reshape(1, INNER),
      g.reshape(1, INNER), b.reshape(1, INNER))


def _final_kernel(x_ref, w_ref, b_ref, g_ref, beta_ref, o_ref):
    x = x_ref[...]
    e = jnp.dot(x, w_ref[...], preferred_element_type=jnp.float32) + b_ref[...]
    n = e.shape[0]
    m = jnp.sum(e, axis=0, keepdims=True) / n
    v = jnp.sum((e - m) ** 2, axis=0, keepdims=True) / n
    e = (e - m) / jnp.sqrt(v + 1e-5) * g_ref[...] + beta_ref[...]
    e = jnp.maximum(e, 0.0)
    B = o_ref.shape[0]
    e = e.reshape(B, n // B, e.shape[-1])
    o_ref[...] = jnp.max(e, axis=1)


def _final_stage(tokens_t, params):
    B, Tt, C = tokens_t.shape
    x = tokens_t.reshape(B * Tt, C)
    return pl.pallas_call(
        _final_kernel,
        out_shape=jax.ShapeDtypeStruct((B, EMB), jnp.float32),
    )(x, params['emb_w'], params['emb_b'].reshape(1, EMB),
      params['emb_g'].reshape(1, EMB), params['emb_beta'].reshape(1, EMB))


def kernel(tokens, centers, lrfs, params):
    B, T, C = tokens.shape
    tokens_t = tokens
    ccoords = tuple(
        centers[..., c].reshape(B, T // 128, 128) for c in range(3))
    T_t = T
    for i in range(NUM_BLOCKS):
        Tq = NUM_CENTER_PTS[i]
        k = NUM_NNS[i]
        rep_idx, qcoords = _fps_stage(ccoords, Tq)
        tokens_q = _gather_dim1(tokens_t, rep_idx)
        if i > 0:
            stats1 = _stats_stage([tokens_q.reshape(B * Tq, C),
                                   tokens_t.reshape(B * T_t, C)])
            g1, b1 = params['bn1_g%d' % i], params['bn1_b%d' % i]
        else:
            stats1, g1, b1 = None, None, None
        Q, Kx, Vx = _prep_stage(tokens_q, tokens_t,
                                params['Wq%d' % i], params['Wkv%d' % i],
                                stats1, g1, b1)
        tcoords_lane = tuple(cc.reshape(B, 1, T_t) for cc in ccoords)
        attn = _attn_stage(qcoords, tcoords_lane, Q, Kx, Vx, tokens_q, k)
        x = attn.reshape(B * Tq, C)
        stats2 = _stats_stage([x])
        y = _tail_stage(x, stats2,
                        params['mlp_w1_%d' % i], params['mlp_b1_%d' % i],
                        params['mlp_w2_%d' % i], params['mlp_b2_%d' % i],
                        params['bn2_g%d' % i], params['bn2_b%d' % i])
        tokens_t = y.reshape(B, Tq, C)
        ccoords = tuple(qc.reshape(B, Tq // 128, 128) for qc in qcoords)
        T_t = Tq
    return _final_stage(tokens_t, params)


# speedup vs baseline: 16.2285x; 1.0216x over previous
import functools

import jax
import jax.numpy as jnp
from jax.experimental import pallas as pl
from jax.experimental.pallas import tpu as pltpu

NUM_BLOCKS = 3
INNER = 128
EMB = 256
NUM_CENTER_PTS = [2048, 1024, 512]
NUM_NNS = [16, 32, 64]
QT = 512


def _gather_dim1(x, idx):
    return jax.vmap(lambda xb, ib: xb[ib])(x, idx)


def _fps_kernel(K, cx_ref, cy_ref, cz_ref, idx_ref, qx_ref, qy_ref, qz_ref):
    x = cx_ref[...]
    y = cy_ref[...]
    z = cz_ref[...]
    B, S, L = x.shape
    lin = (jax.lax.broadcasted_iota(jnp.int32, (1, S, L), 1) * L
           + jax.lax.broadcasted_iota(jnp.int32, (1, S, L), 2))
    lastx = x[:, 0:1, 0:1]
    lasty = y[:, 0:1, 0:1]
    lastz = z[:, 0:1, 0:1]
    idx_ref[:, 0:1, :] = jnp.zeros((B, 1, 1), jnp.int32)
    qx_ref[:, 0:1, :] = lastx
    qy_ref[:, 0:1, :] = lasty
    qz_ref[:, 0:1, :] = lastz
    d0 = jnp.full((B, S, L), jnp.inf, dtype=jnp.float32)

    def body(i, st):
        d, lx, ly, lz = st
        dx = x - lx
        dy = y - ly
        dz = z - lz
        dn = dx * dx + dy * dy + dz * dz
        d = jnp.minimum(d, dn)
        m = jnp.max(d, axis=(1, 2), keepdims=True)
        cand = jnp.where(d == m, lin, jnp.int32(2147483647))
        nxt = jnp.min(cand, axis=(1, 2), keepdims=True)
        hot = lin == nxt
        lx = jnp.sum(jnp.where(hot, x, 0.0), axis=(1, 2), keepdims=True)
        ly = jnp.sum(jnp.where(hot, y, 0.0), axis=(1, 2), keepdims=True)
        lz = jnp.sum(jnp.where(hot, z, 0.0), axis=(1, 2), keepdims=True)
        idx_ref[:, pl.ds(i, 1), :] = nxt
        qx_ref[:, pl.ds(i, 1), :] = lx
        qy_ref[:, pl.ds(i, 1), :] = ly
        qz_ref[:, pl.ds(i, 1), :] = lz
        return d, lx, ly, lz

    jax.lax.fori_loop(1, K, body, (d0, lastx, lasty, lastz))


def _fps_stage(ccoords, K):
    cx, cy, cz = ccoords
    B = cx.shape[0]
    out_shapes = (
        jax.ShapeDtypeStruct((B, K, 1), jnp.int32),
        jax.ShapeDtypeStruct((B, K, 1), jnp.float32),
        jax.ShapeDtypeStruct((B, K, 1), jnp.float32),
        jax.ShapeDtypeStruct((B, K, 1), jnp.float32),
    )
    idx, qx, qy, qz = pl.pallas_call(
        functools.partial(_fps_kernel, K),
        out_shape=out_shapes,
    )(cx, cy, cz)
    return idx[..., 0], (qx, qy, qz)


def _stats_kernel(*refs):
    o_ref = refs[-1]
    s = jnp.zeros((1, INNER), jnp.float32)
    q = jnp.zeros((1, INNER), jnp.float32)
    for r in refs[:-1]:
        x = r[...]
        s = s + jnp.sum(x, axis=0, keepdims=True)
        q = q + jnp.sum(x * x, axis=0, keepdims=True)
    o_ref[...] = jnp.concatenate(
        [s, q, jnp.zeros((6, INNER), jnp.float32)], axis=0)


def _stats_stage(arrays):
    return pl.pallas_call(
        _stats_kernel,
        out_shape=jax.ShapeDtypeStruct((8, INNER), jnp.float32),
    )(*arrays)


def _prep_kernel(nrows, tq_ref, tt_ref, wq_ref, wkv_ref, st_ref, g_ref,
                 b_ref, q_ref, k_ref, v_ref):
    tq = tq_ref[0]
    tt = tt_ref[0]
    if st_ref is not None:
        mean = st_ref[0:1, :] / nrows
        var = st_ref[1:2, :] / nrows - mean * mean
        inv = jax.lax.rsqrt(var + 1e-5) * g_ref[...]
        shift = b_ref[...] - mean * inv
        tq = tq * inv + shift
        tt = tt * inv + shift
    q_ref[0] = jnp.dot(tq, wq_ref[...], preferred_element_type=jnp.float32)
    kv = jnp.dot(tt, wkv_ref[...], preferred_element_type=jnp.float32)
    k_ref[0] = kv[:, :INNER]
    v_ref[0] = kv[:, INNER:]


def _prep_stage(tokens_q, tokens_t, wq, wkv, stats, g, b):
    B, K, _ = tokens_q.shape
    T = tokens_t.shape[1]
    has_bn = stats is not None
    nrows = B * (K + T)
    in_specs = [
        pl.BlockSpec((1, K, INNER), lambda i: (i, 0, 0)),
        pl.BlockSpec((1, T, INNER), lambda i: (i, 0, 0)),
        pl.BlockSpec((INNER, INNER), lambda i: (0, 0)),
        pl.BlockSpec((INNER, 2 * INNER), lambda i: (0, 0)),
    ]
    args = [tokens_q, tokens_t, wq, wkv]
    if has_bn:
        in_specs += [
            pl.BlockSpec((8, INNER), lambda i: (0, 0)),
            pl.BlockSpec((1, INNER), lambda i: (0, 0)),
            pl.BlockSpec((1, INNER), lambda i: (0, 0)),
        ]
        args += [stats, g.reshape(1, INNER), b.reshape(1, INNER)]
        body = functools.partial(_prep_kernel, nrows)
    else:
        body = (lambda tqr, ttr, wqr, wkvr, qr, kr, vr:
                _prep_kernel(nrows, tqr, ttr, wqr, wkvr, None, None, None,
                             qr, kr, vr))
    out_shapes = (
        jax.ShapeDtypeStruct((B, K, INNER), jnp.float32),
        jax.ShapeDtypeStruct((B, T, INNER), jnp.float32),
        jax.ShapeDtypeStruct((B, T, INNER), jnp.float32),
    )
    out_specs = (
        pl.BlockSpec((1, K, INNER), lambda i: (i, 0, 0)),
        pl.BlockSpec((1, T, INNER), lambda i: (i, 0, 0)),
        pl.BlockSpec((1, T, INNER), lambda i: (i, 0, 0)),
    )
    return pl.pallas_call(
        body, grid=(B,), in_specs=in_specs, out_specs=out_specs,
        out_shape=out_shapes,
    )(*args)


def _attn_kernel(k, scale, qx_ref, qy_ref, qz_ref, tx_ref, ty_ref, tz_ref,
                 q_ref, kx_ref, vx_ref, skip_ref, o_ref, dw_ref, nm_ref):
    qx = qx_ref[0]
    qy = qy_ref[0]
    qz = qz_ref[0]
    tx = tx_ref[0]
    ty = ty_ref[0]
    tz = tz_ref[0]
    T = tx.shape[1]
    n_q = qx.shape[0]
    sq = qx * qx + qy * qy + qz * qz
    st = tx * tx + ty * ty + tz * tz
    qt = qx * tx + qy * ty + qz * tz
    d = (sq + st) - 2.0 * qt
    bits = jax.lax.bitcast_convert_type(d, jnp.int32)
    dw_ref[...] = jnp.where(bits < 0, bits ^ jnp.int32(0x7FFFFFFF), bits)

    def rbody(i, X):
        shift = jax.lax.shift_left(jnp.int32(1), jnp.int32(31) - i)
        c = X + shift
        cnt = jnp.sum((dw_ref[...] < c).astype(jnp.int32),
                      axis=1, keepdims=True)
        return jnp.where(cnt < k, c, X)

    X = jax.lax.fori_loop(
        0, 32, rbody, jnp.full((n_q, 1), jnp.int32(-2147483648)))
    skey = dw_ref[...]
    less = skey < X
    cnt_less = jnp.sum(less.astype(jnp.int32), axis=1, keepdims=True)
    eq = (skey == X).astype(jnp.int32)
    s = eq
    sh = 1
    while sh < T:
        s = s + jnp.concatenate(
            [jnp.zeros((n_q, sh), jnp.int32), s[:, :T - sh]], axis=1)
        sh *= 2
    sel = less | ((eq == 1) & (s <= (k - cnt_less)))
    nm_ref[...] = jnp.where(sel, 0.0, -1e30)
    qk = jax.lax.dot_general(
        q_ref[0], kx_ref[0], (((1,), (1,)), ((), ())),
        preferred_element_type=jnp.float32)
    logits = qk * scale + nm_ref[...]
    mx = jnp.max(logits, axis=1, keepdims=True)
    e = jnp.exp(logits - mx)
    a = e / jnp.sum(e, axis=1, keepdims=True)
    av = jnp.dot(a, vx_ref[0], preferred_element_type=jnp.float32)
    o_ref[0] = av + skip_ref[0]


def _attn_stage(qcoords, tcoords_lane, Q, Kx, Vx, skip, k):
    B, K, _ = Q.shape
    T = Kx.shape[1]
    scale = 1.0 / (float(INNER) ** 0.5)
    qspec = pl.BlockSpec((1, QT, 1), lambda b, t: (b, t, 0))
    tspec = pl.BlockSpec((1, 1, T), lambda b, t: (b, 0, 0))
    in_specs = [qspec, qspec, qspec, tspec, tspec, tspec,
                pl.BlockSpec((1, QT, INNER), lambda b, t: (b, t, 0)),
                pl.BlockSpec((1, T, INNER), lambda b, t: (b, 0, 0)),
                pl.BlockSpec((1, T, INNER), lambda b, t: (b, 0, 0)),
                pl.BlockSpec((1, QT, INNER), lambda b, t: (b, t, 0))]
    return pl.pallas_call(
        functools.partial(_attn_kernel, k, scale),
        grid=(B, K // QT),
        in_specs=in_specs,
        out_specs=pl.BlockSpec((1, QT, INNER), lambda b, t: (b, t, 0)),
        out_shape=jax.ShapeDtypeStruct((B, K, INNER), jnp.float32),
        scratch_shapes=[pltpu.VMEM((QT, T), jnp.int32),
                        pltpu.VMEM((QT, T), jnp.float32)],
    )(*qcoords, *tcoords_lane, Q, Kx, Vx, skip)


def _tail_kernel(nrows, x_ref, st_ref, w1_ref, b1_ref, w2_ref, b2_ref,
                 g_ref, b_ref, o_ref):
    x = x_ref[...]
    mean = st_ref[0:1, :] / nrows
    var = st_ref[1:2, :] / nrows - mean * mean
    inv = jax.lax.rsqrt(var + 1e-5) * g_ref[...]
    shift = b_ref[...] - mean * inv
    xn = x * inv + shift
    h = jnp.maximum(
        jnp.dot(xn, w1_ref[...], preferred_element_type=jnp.float32)
        + b1_ref[...], 0.0)
    y = (jnp.dot(h, w2_ref[...], preferred_element_type=jnp.float32)
         + b2_ref[...])
    o_ref[...] = y + x


def _tail_stage(x, stats, w1, b1, w2, b2, g, b):
    N = x.shape[0]
    R = 1024
    in_specs = [
        pl.BlockSpec((R, INNER), lambda i: (i, 0)),
        pl.BlockSpec((8, INNER), lambda i: (0, 0)),
        pl.BlockSpec((INNER, 2 * INNER), lambda i: (0, 0)),
        pl.BlockSpec((1, 2 * INNER), lambda i: (0, 0)),
        pl.BlockSpec((2 * INNER, INNER), lambda i: (0, 0)),
        pl.BlockSpec((1, INNER), lambda i: (0, 0)),
        pl.BlockSpec((1, INNER), lambda i: (0, 0)),
        pl.BlockSpec((1, INNER), lambda i: (0, 0)),
    ]
    return pl.pallas_call(
        functools.partial(_tail_kernel, N),
        grid=(N // R,),
        in_specs=in_specs,
        out_specs=pl.BlockSpec((R, INNER), lambda i: (i, 0)),
        out_shape=jax.ShapeDtypeStruct((N, INNER), jnp.float32),
    )(x, stats, w1, b1.reshape(1, 2 * INNER), w2, b2.reshape(1, INNER),
      g.reshape(1, INNER), b.reshape(1, INNER))


def _final_kernel(x_ref, w_ref, b_ref, g_ref, beta_ref, o_ref):
    x = x_ref[...]
    e = jnp.dot(x, w_ref[...], preferred_element_type=jnp.float32) + b_ref[...]
    n = e.shape[0]
    m = jnp.sum(e, axis=0, keepdims=True) / n
    v = jnp.sum((e - m) ** 2, axis=0, keepdims=True) / n
    e = (e - m) / jnp.sqrt(v + 1e-5) * g_ref[...] + beta_ref[...]
    e = jnp.maximum(e, 0.0)
    B = o_ref.shape[0]
    e = e.reshape(B, n // B, e.shape[-1])
    o_ref[...] = jnp.max(e, axis=1)


def _final_stage(tokens_t, params):
    B, Tt, C = tokens_t.shape
    x = tokens_t.reshape(B * Tt, C)
    return pl.pallas_call(
        _final_kernel,
        out_shape=jax.ShapeDtypeStruct((B, EMB), jnp.float32),
    )(x, params['emb_w'], params['emb_b'].reshape(1, EMB),
      params['emb_g'].reshape(1, EMB), params['emb_beta'].reshape(1, EMB))


def kernel(tokens, centers, lrfs, params):
    B, T, C = tokens.shape
    tokens_t = tokens
    ccoords = tuple(
        centers[..., c].reshape(B, T // 128, 128) for c in range(3))
    T_t = T
    for i in range(NUM_BLOCKS):
        Tq = NUM_CENTER_PTS[i]
        k = NUM_NNS[i]
        rep_idx, qcoords = _fps_stage(ccoords, Tq)
        tokens_q = _gather_dim1(tokens_t, rep_idx)
        if i > 0:
            stats1 = _stats_stage([tokens_q.reshape(B * Tq, C),
                                   tokens_t.reshape(B * T_t, C)])
            g1, b1 = params['bn1_g%d' % i], params['bn1_b%d' % i]
        else:
            stats1, g1, b1 = None, None, None
        Q, Kx, Vx = _prep_stage(tokens_q, tokens_t,
                                params['Wq%d' % i], params['Wkv%d' % i],
                                stats1, g1, b1)
        tcoords_lane = tuple(cc.reshape(B, 1, T_t) for cc in ccoords)
        attn = _attn_stage(qcoords, tcoords_lane, Q, Kx, Vx, tokens_q, k)
        x = attn.reshape(B * Tq, C)
        stats2 = _stats_stage([x])
        y = _tail_stage(x, stats2,
                        params['mlp_w1_%d' % i], params['mlp_b1_%d' % i],
                        params['mlp_w2_%d' % i], params['mlp_b2_%d' % i],
                        params['bn2_g%d' % i], params['bn2_b%d' % i])
        tokens_t = y.reshape(B, Tq, C)
        ccoords = tuple(qc.reshape(B, Tq // 128, 128) for qc in qcoords)
        T_t = Tq
    return _final_stage(tokens_t, params)


# speedup vs baseline: 16.3497x; 1.0075x over previous
import functools

import jax
import jax.numpy as jnp
from jax.experimental import pallas as pl
from jax.experimental.pallas import tpu as pltpu
from jax.experimental.pallas import tpu_sc as plsc

NUM_BLOCKS = 3
INNER = 128
EMB = 256
NUM_CENTER_PTS = [2048, 1024, 512]
NUM_NNS = [16, 32, 64]
QT = 512


def _gather_dim1(x, idx):
    return jax.vmap(lambda xb, ib: xb[ib])(x, idx)


def _sc_gather_rows(table, idx):
    info = plsc.get_sparse_core_info()
    nw = info.num_cores * info.num_subcores
    M = idx.shape[0]
    b_per_w = M // nw
    mesh = plsc.VectorSubcoreMesh(core_axis_name="c", subcore_axis_name="s")

    @functools.partial(
        pl.kernel, mesh=mesh,
        out_type=jax.ShapeDtypeStruct((M, INNER), jnp.float32),
        scratch_types=[
            pltpu.VMEM((b_per_w,), jnp.int32),
            pltpu.VMEM((b_per_w, INNER), jnp.float32),
            pltpu.SemaphoreType.DMA,
        ],
    )
    def gk(table_hbm, idx_hbm, out_hbm, idx_v, rows_v, sem):
        wid = (jax.lax.axis_index("s") * info.num_cores
               + jax.lax.axis_index("c"))
        base = wid * b_per_w
        pltpu.sync_copy(idx_hbm.at[pl.ds(base, b_per_w)], idx_v)
        pltpu.async_copy(table_hbm.at[idx_v], rows_v, sem).wait()
        pltpu.sync_copy(rows_v, out_hbm.at[pl.ds(base, b_per_w)])

    return gk(table, idx)


def _fps_kernel(K, cx_ref, cy_ref, cz_ref, idx_ref, qx_ref, qy_ref, qz_ref):
    x = cx_ref[...]
    y = cy_ref[...]
    z = cz_ref[...]
    B, S, L = x.shape
    lin = (jax.lax.broadcasted_iota(jnp.int32, (1, S, L), 1) * L
           + jax.lax.broadcasted_iota(jnp.int32, (1, S, L), 2))
    lastx = x[:, 0:1, 0:1]
    lasty = y[:, 0:1, 0:1]
    lastz = z[:, 0:1, 0:1]
    idx_ref[:, 0:1, :] = jnp.zeros((B, 1, 1), jnp.int32)
    qx_ref[:, 0:1, :] = lastx
    qy_ref[:, 0:1, :] = lasty
    qz_ref[:, 0:1, :] = lastz
    d0 = jnp.full((B, S, L), jnp.inf, dtype=jnp.float32)

    def body(i, st):
        d, lx, ly, lz = st
        dx = x - lx
        dy = y - ly
        dz = z - lz
        dn = dx * dx + dy * dy + dz * dz
        d = jnp.minimum(d, dn)
        m = jnp.max(d, axis=(1, 2), keepdims=True)
        cand = jnp.where(d == m, lin, jnp.int32(2147483647))
        nxt = jnp.min(cand, axis=(1, 2), keepdims=True)
        hot = lin == nxt
        lx = jnp.sum(jnp.where(hot, x, 0.0), axis=(1, 2), keepdims=True)
        ly = jnp.sum(jnp.where(hot, y, 0.0), axis=(1, 2), keepdims=True)
        lz = jnp.sum(jnp.where(hot, z, 0.0), axis=(1, 2), keepdims=True)
        idx_ref[:, pl.ds(i, 1), :] = nxt
        qx_ref[:, pl.ds(i, 1), :] = lx
        qy_ref[:, pl.ds(i, 1), :] = ly
        qz_ref[:, pl.ds(i, 1), :] = lz
        return d, lx, ly, lz

    jax.lax.fori_loop(1, K, body, (d0, lastx, lasty, lastz))


def _fps_stage(ccoords, K):
    cx, cy, cz = ccoords
    B = cx.shape[0]
    out_shapes = (
        jax.ShapeDtypeStruct((B, K, 1), jnp.int32),
        jax.ShapeDtypeStruct((B, K, 1), jnp.float32),
        jax.ShapeDtypeStruct((B, K, 1), jnp.float32),
        jax.ShapeDtypeStruct((B, K, 1), jnp.float32),
    )
    idx, qx, qy, qz = pl.pallas_call(
        functools.partial(_fps_kernel, K),
        out_shape=out_shapes,
    )(cx, cy, cz)
    return idx[..., 0], (qx, qy, qz)


def _stats_kernel(*refs):
    o_ref = refs[-1]
    s = jnp.zeros((1, INNER), jnp.float32)
    q = jnp.zeros((1, INNER), jnp.float32)
    for r in refs[:-1]:
        x = r[...]
        s = s + jnp.sum(x, axis=0, keepdims=True)
        q = q + jnp.sum(x * x, axis=0, keepdims=True)
    o_ref[...] = jnp.concatenate(
        [s, q, jnp.zeros((6, INNER), jnp.float32)], axis=0)


def _stats_stage(arrays):
    return pl.pallas_call(
        _stats_kernel,
        out_shape=jax.ShapeDtypeStruct((8, INNER), jnp.float32),
    )(*arrays)


def _prep_kernel(nrows, tq_ref, tt_ref, wq_ref, wkv_ref, st_ref, g_ref,
                 b_ref, q_ref, k_ref, v_ref):
    tq = tq_ref[0]
    tt = tt_ref[0]
    if st_ref is not None:
        mean = st_ref[0:1, :] / nrows
        var = st_ref[1:2, :] / nrows - mean * mean
        inv = jax.lax.rsqrt(var + 1e-5) * g_ref[...]
        shift = b_ref[...] - mean * inv
        tq = tq * inv + shift
        tt = tt * inv + shift
    q_ref[0] = jnp.dot(tq, wq_ref[...], preferred_element_type=jnp.float32)
    kv = jnp.dot(tt, wkv_ref[...], preferred_element_type=jnp.float32)
    k_ref[0] = kv[:, :INNER]
    v_ref[0] = kv[:, INNER:]


def _prep_stage(tokens_q, tokens_t, wq, wkv, stats, g, b):
    B, K, _ = tokens_q.shape
    T = tokens_t.shape[1]
    has_bn = stats is not None
    nrows = B * (K + T)
    in_specs = [
        pl.BlockSpec((1, K, INNER), lambda i: (i, 0, 0)),
        pl.BlockSpec((1, T, INNER), lambda i: (i, 0, 0)),
        pl.BlockSpec((INNER, INNER), lambda i: (0, 0)),
        pl.BlockSpec((INNER, 2 * INNER), lambda i: (0, 0)),
    ]
    args = [tokens_q, tokens_t, wq, wkv]
    if has_bn:
        in_specs += [
            pl.BlockSpec((8, INNER), lambda i: (0, 0)),
            pl.BlockSpec((1, INNER), lambda i: (0, 0)),
            pl.BlockSpec((1, INNER), lambda i: (0, 0)),
        ]
        args += [stats, g.reshape(1, INNER), b.reshape(1, INNER)]
        body = functools.partial(_prep_kernel, nrows)
    else:
        body = (lambda tqr, ttr, wqr, wkvr, qr, kr, vr:
                _prep_kernel(nrows, tqr, ttr, wqr, wkvr, None, None, None,
                             qr, kr, vr))
    out_shapes = (
        jax.ShapeDtypeStruct((B, K, INNER), jnp.float32),
        jax.ShapeDtypeStruct((B, T, INNER), jnp.float32),
        jax.ShapeDtypeStruct((B, T, INNER), jnp.float32),
    )
    out_specs = (
        pl.BlockSpec((1, K, INNER), lambda i: (i, 0, 0)),
        pl.BlockSpec((1, T, INNER), lambda i: (i, 0, 0)),
        pl.BlockSpec((1, T, INNER), lambda i: (i, 0, 0)),
    )
    return pl.pallas_call(
        body, grid=(B,), in_specs=in_specs, out_specs=out_specs,
        out_shape=out_shapes,
    )(*args)


def _attn_kernel(k, scale, qx_ref, qy_ref, qz_ref, tx_ref, ty_ref, tz_ref,
                 q_ref, kx_ref, vx_ref, skip_ref, o_ref, dw_ref, nm_ref):
    qx = qx_ref[0]
    qy = qy_ref[0]
    qz = qz_ref[0]
    tx = tx_ref[0]
    ty = ty_ref[0]
    tz = tz_ref[0]
    T = tx.shape[1]
    n_q = qx.shape[0]
    sq = qx * qx + qy * qy + qz * qz
    st = tx * tx + ty * ty + tz * tz
    qt = qx * tx + qy * ty + qz * tz
    d = (sq + st) - 2.0 * qt
    bits = jax.lax.bitcast_convert_type(d, jnp.int32)
    dw_ref[...] = jnp.where(bits < 0, bits ^ jnp.int32(0x7FFFFFFF), bits)

    def rbody(i, X):
        shift = jax.lax.shift_left(jnp.int32(1), jnp.int32(31) - i)
        c = X + shift
        cnt = jnp.sum((dw_ref[...] < c).astype(jnp.int32),
                      axis=1, keepdims=True)
        return jnp.where(cnt < k, c, X)

    X = jax.lax.fori_loop(
        0, 32, rbody, jnp.full((n_q, 1), jnp.int32(-2147483648)))
    skey = dw_ref[...]
    less = skey < X
    cnt_less = jnp.sum(less.astype(jnp.int32), axis=1, keepdims=True)
    eq = (skey == X).astype(jnp.int32)
    s = eq
    sh = 1
    while sh < T:
        s = s + jnp.concatenate(
            [jnp.zeros((n_q, sh), jnp.int32), s[:, :T - sh]], axis=1)
        sh *= 2
    sel = less | ((eq == 1) & (s <= (k - cnt_less)))
    nm_ref[...] = jnp.where(sel, 0.0, -1e30)
    qk = jax.lax.dot_general(
        q_ref[0], kx_ref[0], (((1,), (1,)), ((), ())),
        preferred_element_type=jnp.float32)
    logits = qk * scale + nm_ref[...]
    mx = jnp.max(logits, axis=1, keepdims=True)
    e = jnp.exp(logits - mx)
    a = e / jnp.sum(e, axis=1, keepdims=True)
    av = jnp.dot(a, vx_ref[0], preferred_element_type=jnp.float32)
    o_ref[0] = av + skip_ref[0]


def _attn_stage(qcoords, tcoords_lane, Q, Kx, Vx, skip, k):
    B, K, _ = Q.shape
    T = Kx.shape[1]
    scale = 1.0 / (float(INNER) ** 0.5)
    qspec = pl.BlockSpec((1, QT, 1), lambda b, t: (b, t, 0))
    tspec = pl.BlockSpec((1, 1, T), lambda b, t: (b, 0, 0))
    in_specs = [qspec, qspec, qspec, tspec, tspec, tspec,
                pl.BlockSpec((1, QT, INNER), lambda b, t: (b, t, 0)),
                pl.BlockSpec((1, T, INNER), lambda b, t: (b, 0, 0)),
                pl.BlockSpec((1, T, INNER), lambda b, t: (b, 0, 0)),
                pl.BlockSpec((1, QT, INNER), lambda b, t: (b, t, 0))]
    return pl.pallas_call(
        functools.partial(_attn_kernel, k, scale),
        grid=(B, K // QT),
        in_specs=in_specs,
        out_specs=pl.BlockSpec((1, QT, INNER), lambda b, t: (b, t, 0)),
        out_shape=jax.ShapeDtypeStruct((B, K, INNER), jnp.float32),
        scratch_shapes=[pltpu.VMEM((QT, T), jnp.int32),
                        pltpu.VMEM((QT, T), jnp.float32)],
    )(*qcoords, *tcoords_lane, Q, Kx, Vx, skip)


def _tail_kernel(nrows, x_ref, st_ref, w1_ref, b1_ref, w2_ref, b2_ref,
                 g_ref, b_ref, o_ref):
    x = x_ref[...]
    mean = st_ref[0:1, :] / nrows
    var = st_ref[1:2, :] / nrows - mean * mean
    inv = jax.lax.rsqrt(var + 1e-5) * g_ref[...]
    shift = b_ref[...] - mean * inv
    xn = x * inv + shift
    h = jnp.maximum(
        jnp.dot(xn, w1_ref[...], preferred_element_type=jnp.float32)
        + b1_ref[...], 0.0)
    y = (jnp.dot(h, w2_ref[...], preferred_element_type=jnp.float32)
         + b2_ref[...])
    o_ref[...] = y + x


def _tail_stage(x, stats, w1, b1, w2, b2, g, b):
    N = x.shape[0]
    R = 1024
    in_specs = [
        pl.BlockSpec((R, INNER), lambda i: (i, 0)),
        pl.BlockSpec((8, INNER), lambda i: (0, 0)),
        pl.BlockSpec((INNER, 2 * INNER), lambda i: (0, 0)),
        pl.BlockSpec((1, 2 * INNER), lambda i: (0, 0)),
        pl.BlockSpec((2 * INNER, INNER), lambda i: (0, 0)),
        pl.BlockSpec((1, INNER), lambda i: (0, 0)),
        pl.BlockSpec((1, INNER), lambda i: (0, 0)),
        pl.BlockSpec((1, INNER), lambda i: (0, 0)),
    ]
    return pl.pallas_call(
        functools.partial(_tail_kernel, N),
        grid=(N // R,),
        in_specs=in_specs,
        out_specs=pl.BlockSpec((R, INNER), lambda i: (i, 0)),
        out_shape=jax.ShapeDtypeStruct((N, INNER), jnp.float32),
    )(x, stats, w1, b1.reshape(1, 2 * INNER), w2, b2.reshape(1, INNER),
      g.reshape(1, INNER), b.reshape(1, INNER))


def _final_kernel(x_ref, w_ref, b_ref, g_ref, beta_ref, o_ref):
    x = x_ref[...]
    e = jnp.dot(x, w_ref[...], preferred_element_type=jnp.float32) + b_ref[...]
    n = e.shape[0]
    m = jnp.sum(e, axis=0, keepdims=True) / n
    v = jnp.sum((e - m) ** 2, axis=0, keepdims=True) / n
    e = (e - m) / jnp.sqrt(v + 1e-5) * g_ref[...] + beta_ref[...]
    e = jnp.maximum(e, 0.0)
    B = o_ref.shape[0]
    e = e.reshape(B, n // B, e.shape[-1])
    o_ref[...] = jnp.max(e, axis=1)


def _final_stage(tokens_t, params):
    B, Tt, C = tokens_t.shape
    x = tokens_t.reshape(B * Tt, C)
    return pl.pallas_call(
        _final_kernel,
        out_shape=jax.ShapeDtypeStruct((B, EMB), jnp.float32),
    )(x, params['emb_w'], params['emb_b'].reshape(1, EMB),
      params['emb_g'].reshape(1, EMB), params['emb_beta'].reshape(1, EMB))


def kernel(tokens, centers, lrfs, params):
    B, T, C = tokens.shape
    tokens_t = tokens
    ccoords = tuple(
        centers[..., c].reshape(B, T // 128, 128) for c in range(3))
    T_t = T
    for i in range(NUM_BLOCKS):
        Tq = NUM_CENTER_PTS[i]
        k = NUM_NNS[i]
        rep_idx, qcoords = _fps_stage(ccoords, Tq)
        flat_idx = (rep_idx
                    + jnp.arange(B, dtype=jnp.int32)[:, None] * T_t
                    ).reshape(B * Tq)
        tokens_q = _sc_gather_rows(
            tokens_t.reshape(B * T_t, C), flat_idx).reshape(B, Tq, C)
        if i > 0:
            stats1 = _stats_stage([tokens_q.reshape(B * Tq, C),
                                   tokens_t.reshape(B * T_t, C)])
            g1, b1 = params['bn1_g%d' % i], params['bn1_b%d' % i]
        else:
            stats1, g1, b1 = None, None, None
        Q, Kx, Vx = _prep_stage(tokens_q, tokens_t,
                                params['Wq%d' % i], params['Wkv%d' % i],
                                stats1, g1, b1)
        tcoords_lane = tuple(cc.reshape(B, 1, T_t) for cc in ccoords)
        attn = _attn_stage(qcoords, tcoords_lane, Q, Kx, Vx, tokens_q, k)
        x = attn.reshape(B * Tq, C)
        stats2 = _stats_stage([x])
        y = _tail_stage(x, stats2,
                        params['mlp_w1_%d' % i], params['mlp_b1_%d' % i],
                        params['mlp_w2_%d' % i], params['mlp_b2_%d' % i],
                        params['bn2_g%d' % i], params['bn2_b%d' % i])
        tokens_t = y.reshape(B, Tq, C)
        ccoords = tuple(qc.reshape(B, Tq // 128, 128) for qc in qcoords)
        T_t = Tq
    return _final_stage(tokens_t, params)


# speedup vs baseline: 16.4976x; 1.0090x over previous
import functools

import jax
import jax.numpy as jnp
from jax.experimental import pallas as pl
from jax.experimental.pallas import tpu as pltpu
from jax.experimental.pallas import tpu_sc as plsc

NUM_BLOCKS = 3
INNER = 128
EMB = 256
NUM_CENTER_PTS = [2048, 1024, 512]
NUM_NNS = [16, 32, 64]
QT = 512


def _gather_dim1(x, idx):
    return jax.vmap(lambda xb, ib: xb[ib])(x, idx)


def _sc_gather_rows(table, idx):
    info = plsc.get_sparse_core_info()
    nw = info.num_cores * info.num_subcores
    M = idx.shape[0]
    b_per_w = M // nw
    mesh = plsc.VectorSubcoreMesh(core_axis_name="c", subcore_axis_name="s")

    @functools.partial(
        pl.kernel, mesh=mesh,
        out_type=jax.ShapeDtypeStruct((M, INNER), jnp.float32),
        scratch_types=[
            pltpu.VMEM((b_per_w,), jnp.int32),
            pltpu.VMEM((b_per_w, INNER), jnp.float32),
            pltpu.SemaphoreType.DMA,
        ],
    )
    def gk(table_hbm, idx_hbm, out_hbm, idx_v, rows_v, sem):
        wid = (jax.lax.axis_index("s") * info.num_cores
               + jax.lax.axis_index("c"))
        base = wid * b_per_w
        pltpu.sync_copy(idx_hbm.at[pl.ds(base, b_per_w)], idx_v)
        pltpu.async_copy(table_hbm.at[idx_v], rows_v, sem).wait()
        pltpu.sync_copy(rows_v, out_hbm.at[pl.ds(base, b_per_w)])

    return gk(table, idx)


def _fps_kernel(K, cx_ref, cy_ref, cz_ref, idx_ref, qx_ref, qy_ref, qz_ref):
    x = cx_ref[...]
    y = cy_ref[...]
    z = cz_ref[...]
    B, S, L = x.shape
    lin = (jax.lax.broadcasted_iota(jnp.int32, (1, S, L), 1) * L
           + jax.lax.broadcasted_iota(jnp.int32, (1, S, L), 2))
    lastx = x[:, 0:1, 0:1]
    lasty = y[:, 0:1, 0:1]
    lastz = z[:, 0:1, 0:1]
    idx_ref[:, 0:1, :] = jnp.zeros((B, 1, 1), jnp.int32)
    qx_ref[:, 0:1, :] = lastx
    qy_ref[:, 0:1, :] = lasty
    qz_ref[:, 0:1, :] = lastz
    d0 = jnp.full((B, S, L), jnp.inf, dtype=jnp.float32)

    def body(i, st):
        d, lx, ly, lz = st
        dx = x - lx
        dy = y - ly
        dz = z - lz
        dn = dx * dx + dy * dy + dz * dz
        d = jnp.minimum(d, dn)
        m = jnp.max(d, axis=(1, 2), keepdims=True)
        cand = jnp.where(d == m, lin, jnp.int32(2147483647))
        nxt = jnp.min(cand, axis=(1, 2), keepdims=True)
        hot = lin == nxt
        lx = jnp.sum(jnp.where(hot, x, 0.0), axis=(1, 2), keepdims=True)
        ly = jnp.sum(jnp.where(hot, y, 0.0), axis=(1, 2), keepdims=True)
        lz = jnp.sum(jnp.where(hot, z, 0.0), axis=(1, 2), keepdims=True)
        idx_ref[:, pl.ds(i, 1), :] = nxt
        qx_ref[:, pl.ds(i, 1), :] = lx
        qy_ref[:, pl.ds(i, 1), :] = ly
        qz_ref[:, pl.ds(i, 1), :] = lz
        return d, lx, ly, lz

    jax.lax.fori_loop(1, K, body, (d0, lastx, lasty, lastz))


def _fps_stage(ccoords, K):
    cx, cy, cz = ccoords
    B = cx.shape[0]
    out_shapes = (
        jax.ShapeDtypeStruct((B, K, 1), jnp.int32),
        jax.ShapeDtypeStruct((B, K, 1), jnp.float32),
        jax.ShapeDtypeStruct((B, K, 1), jnp.float32),
        jax.ShapeDtypeStruct((B, K, 1), jnp.float32),
    )
    idx, qx, qy, qz = pl.pallas_call(
        functools.partial(_fps_kernel, K),
        out_shape=out_shapes,
    )(cx, cy, cz)
    return idx[..., 0], (qx, qy, qz)


def _stats_kernel(*refs):
    o_ref = refs[-1]
    s = jnp.zeros((1, INNER), jnp.float32)
    q = jnp.zeros((1, INNER), jnp.float32)
    for r in refs[:-1]:
        x = r[...]
        s = s + jnp.sum(x, axis=0, keepdims=True)
        q = q + jnp.sum(x * x, axis=0, keepdims=True)
    o_ref[...] = jnp.concatenate(
        [s, q, jnp.zeros((6, INNER), jnp.float32)], axis=0)


def _stats_stage(arrays):
    return pl.pallas_call(
        _stats_kernel,
        out_shape=jax.ShapeDtypeStruct((8, INNER), jnp.float32),
    )(*arrays)


def _prep_kernel(nrows, tq_ref, tt_ref, wq_ref, wkv_ref, stq_ref, stt_ref,
                 g_ref, b_ref, q_ref, k_ref, v_ref):
    tq = tq_ref[0]
    tt = tt_ref[0]
    if stq_ref is not None:
        mean = (stq_ref[0:1, :] + stt_ref[0:1, :]) / nrows
        var = (stq_ref[1:2, :] + stt_ref[1:2, :]) / nrows - mean * mean
        inv = jax.lax.rsqrt(var + 1e-5) * g_ref[...]
        shift = b_ref[...] - mean * inv
        tq = tq * inv + shift
        tt = tt * inv + shift
    q_ref[0] = jnp.dot(tq, wq_ref[...], preferred_element_type=jnp.float32)
    kv = jnp.dot(tt, wkv_ref[...], preferred_element_type=jnp.float32)
    k_ref[0] = kv[:, :INNER]
    v_ref[0] = kv[:, INNER:]


def _prep_stage(tokens_q, tokens_t, wq, wkv, stats_q, stats_t, g, b):
    B, K, _ = tokens_q.shape
    T = tokens_t.shape[1]
    has_bn = stats_q is not None
    nrows = B * (K + T)
    in_specs = [
        pl.BlockSpec((1, K, INNER), lambda i: (i, 0, 0)),
        pl.BlockSpec((1, T, INNER), lambda i: (i, 0, 0)),
        pl.BlockSpec((INNER, INNER), lambda i: (0, 0)),
        pl.BlockSpec((INNER, 2 * INNER), lambda i: (0, 0)),
    ]
    args = [tokens_q, tokens_t, wq, wkv]
    if has_bn:
        in_specs += [
            pl.BlockSpec((8, INNER), lambda i: (0, 0)),
            pl.BlockSpec((8, INNER), lambda i: (0, 0)),
            pl.BlockSpec((1, INNER), lambda i: (0, 0)),
            pl.BlockSpec((1, INNER), lambda i: (0, 0)),
        ]
        args += [stats_q, stats_t, g.reshape(1, INNER), b.reshape(1, INNER)]
        body = functools.partial(_prep_kernel, nrows)
    else:
        body = (lambda tqr, ttr, wqr, wkvr, qr, kr, vr:
                _prep_kernel(nrows, tqr, ttr, wqr, wkvr, None, None, None,
                             None, qr, kr, vr))
    out_shapes = (
        jax.ShapeDtypeStruct((B, K, INNER), jnp.float32),
        jax.ShapeDtypeStruct((B, T, INNER), jnp.float32),
        jax.ShapeDtypeStruct((B, T, INNER), jnp.float32),
    )
    out_specs = (
        pl.BlockSpec((1, K, INNER), lambda i: (i, 0, 0)),
        pl.BlockSpec((1, T, INNER), lambda i: (i, 0, 0)),
        pl.BlockSpec((1, T, INNER), lambda i: (i, 0, 0)),
    )
    return pl.pallas_call(
        body, grid=(B,), in_specs=in_specs, out_specs=out_specs,
        out_shape=out_shapes,
    )(*args)


def _attn_kernel(k, scale, qx_ref, qy_ref, qz_ref, tx_ref, ty_ref, tz_ref,
                 q_ref, kx_ref, vx_ref, skip_ref, o_ref, st_ref,
                 dw_ref, nm_ref):
    qx = qx_ref[0]
    qy = qy_ref[0]
    qz = qz_ref[0]
    tx = tx_ref[0]
    ty = ty_ref[0]
    tz = tz_ref[0]
    T = tx.shape[1]
    n_q = qx.shape[0]
    sq = qx * qx + qy * qy + qz * qz
    st = tx * tx + ty * ty + tz * tz
    qmat = jnp.concatenate([qx, qy, qz], axis=1)
    tmat = jnp.concatenate([tx, ty, tz], axis=0)
    qt = jnp.dot(qmat, tmat, preferred_element_type=jnp.float32)
    d = (sq + st) - 2.0 * qt
    bits = jax.lax.bitcast_convert_type(d, jnp.int32)
    dw_ref[...] = jnp.where(bits < 0, bits ^ jnp.int32(0x7FFFFFFF), bits)

    def rbody(i, X):
        shift = jax.lax.shift_left(jnp.int32(1), jnp.int32(31) - i)
        c = X + shift
        cnt = jnp.sum((dw_ref[...] < c).astype(jnp.int32),
                      axis=1, keepdims=True)
        return jnp.where(cnt < k, c, X)

    X = jax.lax.fori_loop(
        0, 32, rbody, jnp.full((n_q, 1), jnp.int32(-2147483648)))
    skey = dw_ref[...]
    less = skey < X
    cnt_less = jnp.sum(less.astype(jnp.int32), axis=1, keepdims=True)
    eq = (skey == X).astype(jnp.int32)
    s = eq
    sh = 1
    while sh < T:
        s = s + jnp.concatenate(
            [jnp.zeros((n_q, sh), jnp.int32), s[:, :T - sh]], axis=1)
        sh *= 2
    sel = less | ((eq == 1) & (s <= (k - cnt_less)))
    nm_ref[...] = jnp.where(sel, 0.0, -1e30)
    qk = jax.lax.dot_general(
        q_ref[0], kx_ref[0], (((1,), (1,)), ((), ())),
        preferred_element_type=jnp.float32)
    logits = qk * scale + nm_ref[...]
    mx = jnp.max(logits, axis=1, keepdims=True)
    e = jnp.exp(logits - mx)
    a = e / jnp.sum(e, axis=1, keepdims=True)
    av = jnp.dot(a, vx_ref[0], preferred_element_type=jnp.float32)
    o = av + skip_ref[0]
    o_ref[0] = o

    @pl.when((pl.program_id(0) == 0) & (pl.program_id(1) == 0))
    def _():
        st_ref[...] = jnp.zeros_like(st_ref)

    st_ref[0:1, :] = st_ref[0:1, :] + jnp.sum(o, axis=0, keepdims=True)
    st_ref[1:2, :] = st_ref[1:2, :] + jnp.sum(o * o, axis=0, keepdims=True)


def _attn_stage(qcoords, tcoords_lane, Q, Kx, Vx, skip, k):
    B, K, _ = Q.shape
    T = Kx.shape[1]
    scale = 1.0 / (float(INNER) ** 0.5)
    qspec = pl.BlockSpec((1, QT, 1), lambda b, t: (b, t, 0))
    tspec = pl.BlockSpec((1, 1, T), lambda b, t: (b, 0, 0))
    in_specs = [qspec, qspec, qspec, tspec, tspec, tspec,
                pl.BlockSpec((1, QT, INNER), lambda b, t: (b, t, 0)),
                pl.BlockSpec((1, T, INNER), lambda b, t: (b, 0, 0)),
                pl.BlockSpec((1, T, INNER), lambda b, t: (b, 0, 0)),
                pl.BlockSpec((1, QT, INNER), lambda b, t: (b, t, 0))]
    return pl.pallas_call(
        functools.partial(_attn_kernel, k, scale),
        grid=(B, K // QT),
        in_specs=in_specs,
        out_specs=(pl.BlockSpec((1, QT, INNER), lambda b, t: (b, t, 0)),
                   pl.BlockSpec((8, INNER), lambda b, t: (0, 0))),
        out_shape=(jax.ShapeDtypeStruct((B, K, INNER), jnp.float32),
                   jax.ShapeDtypeStruct((8, INNER), jnp.float32)),
        scratch_shapes=[pltpu.VMEM((QT, T), jnp.int32),
                        pltpu.VMEM((QT, T), jnp.float32)],
    )(*qcoords, *tcoords_lane, Q, Kx, Vx, skip)


def _tail_kernel(nrows, x_ref, st_ref, w1_ref, b1_ref, w2_ref, b2_ref,
                 g_ref, b_ref, o_ref, so_ref):
    x = x_ref[...]
    mean = st_ref[0:1, :] / nrows
    var = st_ref[1:2, :] / nrows - mean * mean
    inv = jax.lax.rsqrt(var + 1e-5) * g_ref[...]
    shift = b_ref[...] - mean * inv
    xn = x * inv + shift
    h = jnp.maximum(
        jnp.dot(xn, w1_ref[...], preferred_element_type=jnp.float32)
        + b1_ref[...], 0.0)
    y = (jnp.dot(h, w2_ref[...], preferred_element_type=jnp.float32)
         + b2_ref[...]) + x
    o_ref[...] = y

    @pl.when(pl.program_id(0) == 0)
    def _():
        so_ref[...] = jnp.zeros_like(so_ref)

    so_ref[0:1, :] = so_ref[0:1, :] + jnp.sum(y, axis=0, keepdims=True)
    so_ref[1:2, :] = so_ref[1:2, :] + jnp.sum(y * y, axis=0, keepdims=True)


def _tail_stage(x, stats, w1, b1, w2, b2, g, b):
    N = x.shape[0]
    R = 1024
    in_specs = [
        pl.BlockSpec((R, INNER), lambda i: (i, 0)),
        pl.BlockSpec((8, INNER), lambda i: (0, 0)),
        pl.BlockSpec((INNER, 2 * INNER), lambda i: (0, 0)),
        pl.BlockSpec((1, 2 * INNER), lambda i: (0, 0)),
        pl.BlockSpec((2 * INNER, INNER), lambda i: (0, 0)),
        pl.BlockSpec((1, INNER), lambda i: (0, 0)),
        pl.BlockSpec((1, INNER), lambda i: (0, 0)),
        pl.BlockSpec((1, INNER), lambda i: (0, 0)),
    ]
    return pl.pallas_call(
        functools.partial(_tail_kernel, N),
        grid=(N // R,),
        in_specs=in_specs,
        out_specs=(pl.BlockSpec((R, INNER), lambda i: (i, 0)),
                   pl.BlockSpec((8, INNER), lambda i: (0, 0))),
        out_shape=(jax.ShapeDtypeStruct((N, INNER), jnp.float32),
                   jax.ShapeDtypeStruct((8, INNER), jnp.float32)),
    )(x, stats, w1, b1.reshape(1, 2 * INNER), w2, b2.reshape(1, INNER),
      g.reshape(1, INNER), b.reshape(1, INNER))


def _final_kernel(x_ref, w_ref, b_ref, g_ref, beta_ref, o_ref):
    x = x_ref[...]
    e = jnp.dot(x, w_ref[...], preferred_element_type=jnp.float32) + b_ref[...]
    n = e.shape[0]
    m = jnp.sum(e, axis=0, keepdims=True) / n
    v = jnp.sum((e - m) ** 2, axis=0, keepdims=True) / n
    e = (e - m) / jnp.sqrt(v + 1e-5) * g_ref[...] + beta_ref[...]
    e = jnp.maximum(e, 0.0)
    B = o_ref.shape[0]
    e = e.reshape(B, n // B, e.shape[-1])
    o_ref[...] = jnp.max(e, axis=1)


def _final_stage(tokens_t, params):
    B, Tt, C = tokens_t.shape
    x = tokens_t.reshape(B * Tt, C)
    return pl.pallas_call(
        _final_kernel,
        out_shape=jax.ShapeDtypeStruct((B, EMB), jnp.float32),
    )(x, params['emb_w'], params['emb_b'].reshape(1, EMB),
      params['emb_g'].reshape(1, EMB), params['emb_beta'].reshape(1, EMB))


def kernel(tokens, centers, lrfs, params):
    B, T, C = tokens.shape
    tokens_t = tokens
    ccoords = tuple(
        centers[..., c].reshape(B, T // 128, 128) for c in range(3))
    T_t = T
    for i in range(NUM_BLOCKS):
        Tq = NUM_CENTER_PTS[i]
        k = NUM_NNS[i]
        rep_idx, qcoords = _fps_stage(ccoords, Tq)
        flat_idx = (rep_idx
                    + jnp.arange(B, dtype=jnp.int32)[:, None] * T_t
                    ).reshape(B * Tq)
        tokens_q = _sc_gather_rows(
            tokens_t.reshape(B * T_t, C), flat_idx).reshape(B, Tq, C)
        if i > 0:
            stats_q = _stats_stage([tokens_q.reshape(B * Tq, C)])
            g1, b1 = params['bn1_g%d' % i], params['bn1_b%d' % i]
        else:
            stats_q, prev_sums, g1, b1 = None, None, None, None
        Q, Kx, Vx = _prep_stage(tokens_q, tokens_t,
                                params['Wq%d' % i], params['Wkv%d' % i],
                                stats_q, prev_sums, g1, b1)
        tcoords_lane = tuple(cc.reshape(B, 1, T_t) for cc in ccoords)
        attn, stats2 = _attn_stage(
            qcoords, tcoords_lane, Q, Kx, Vx, tokens_q, k)
        x = attn.reshape(B * Tq, C)
        y, prev_sums = _tail_stage(
            x, stats2,
            params['mlp_w1_%d' % i], params['mlp_b1_%d' % i],
            params['mlp_w2_%d' % i], params['mlp_b2_%d' % i],
            params['bn2_g%d' % i], params['bn2_b%d' % i])
        tokens_t = y.reshape(B, Tq, C)
        ccoords = tuple(qc.reshape(B, Tq // 128, 128) for qc in qcoords)
        T_t = Tq
    return _final_stage(tokens_t, params)


# speedup vs baseline: 18.8156x; 1.1405x over previous
import functools

import jax
import jax.numpy as jnp
from jax.experimental import pallas as pl
from jax.experimental.pallas import tpu as pltpu
from jax.experimental.pallas import tpu_sc as plsc

NUM_BLOCKS = 3
INNER = 128
EMB = 256
NUM_CENTER_PTS = [2048, 1024, 512]
NUM_NNS = [16, 32, 64]
QT = 512


def _gather_dim1(x, idx):
    return jax.vmap(lambda xb, ib: xb[ib])(x, idx)


def _sc_gather_rows(table, idx):
    info = plsc.get_sparse_core_info()
    nw = info.num_cores * info.num_subcores
    M = idx.shape[0]
    b_per_w = M // nw
    mesh = plsc.VectorSubcoreMesh(core_axis_name="c", subcore_axis_name="s")

    @functools.partial(
        pl.kernel, mesh=mesh,
        out_type=jax.ShapeDtypeStruct((M, INNER), jnp.float32),
        scratch_types=[
            pltpu.VMEM((b_per_w,), jnp.int32),
            pltpu.VMEM((b_per_w, INNER), jnp.float32),
            pltpu.SemaphoreType.DMA,
        ],
    )
    def gk(table_hbm, idx_hbm, out_hbm, idx_v, rows_v, sem):
        wid = (jax.lax.axis_index("s") * info.num_cores
               + jax.lax.axis_index("c"))
        base = wid * b_per_w
        pltpu.sync_copy(idx_hbm.at[pl.ds(base, b_per_w)], idx_v)
        pltpu.async_copy(table_hbm.at[idx_v], rows_v, sem).wait()
        pltpu.sync_copy(rows_v, out_hbm.at[pl.ds(base, b_per_w)])

    return gk(table, idx)


def _fps_kernel(K, cx_ref, cy_ref, cz_ref, idx_ref, qx_ref, qy_ref, qz_ref):
    x = cx_ref[...]
    y = cy_ref[...]
    z = cz_ref[...]
    B, S, L = x.shape
    lin = (jax.lax.broadcasted_iota(jnp.int32, (1, S, L), 1) * L
           + jax.lax.broadcasted_iota(jnp.int32, (1, S, L), 2))
    lastx = x[:, 0:1, 0:1]
    lasty = y[:, 0:1, 0:1]
    lastz = z[:, 0:1, 0:1]
    idx_ref[:, 0:1, :] = jnp.zeros((B, 1, 1), jnp.int32)
    qx_ref[:, 0:1, :] = lastx
    qy_ref[:, 0:1, :] = lasty
    qz_ref[:, 0:1, :] = lastz
    d0 = jnp.full((B, S, L), jnp.inf, dtype=jnp.float32)

    def body(i, st):
        d, lx, ly, lz = st
        dx = x - lx
        dy = y - ly
        dz = z - lz
        dn = dx * dx + dy * dy + dz * dz
        d = jnp.minimum(d, dn)
        m = jnp.max(d, axis=(1, 2), keepdims=True)
        cand = jnp.where(d == m, lin, jnp.int32(2147483647))
        nxt = jnp.min(cand, axis=(1, 2), keepdims=True)
        hot = lin == nxt
        lx = jnp.sum(jnp.where(hot, x, 0.0), axis=(1, 2), keepdims=True)
        ly = jnp.sum(jnp.where(hot, y, 0.0), axis=(1, 2), keepdims=True)
        lz = jnp.sum(jnp.where(hot, z, 0.0), axis=(1, 2), keepdims=True)
        idx_ref[:, pl.ds(i, 1), :] = nxt
        qx_ref[:, pl.ds(i, 1), :] = lx
        qy_ref[:, pl.ds(i, 1), :] = ly
        qz_ref[:, pl.ds(i, 1), :] = lz
        return d, lx, ly, lz

    jax.lax.fori_loop(1, K, body, (d0, lastx, lasty, lastz))


def _fps_stage(ccoords, K):
    cx, cy, cz = ccoords
    B = cx.shape[0]
    out_shapes = (
        jax.ShapeDtypeStruct((B, K, 1), jnp.int32),
        jax.ShapeDtypeStruct((B, K, 1), jnp.float32),
        jax.ShapeDtypeStruct((B, K, 1), jnp.float32),
        jax.ShapeDtypeStruct((B, K, 1), jnp.float32),
    )
    idx, qx, qy, qz = pl.pallas_call(
        functools.partial(_fps_kernel, K),
        out_shape=out_shapes,
    )(cx, cy, cz)
    return idx[..., 0], (qx, qy, qz)


def _stats_kernel(*refs):
    o_ref = refs[-1]
    s = jnp.zeros((1, INNER), jnp.float32)
    q = jnp.zeros((1, INNER), jnp.float32)
    for r in refs[:-1]:
        x = r[...]
        s = s + jnp.sum(x, axis=0, keepdims=True)
        q = q + jnp.sum(x * x, axis=0, keepdims=True)
    o_ref[...] = jnp.concatenate(
        [s, q, jnp.zeros((6, INNER), jnp.float32)], axis=0)


def _stats_stage(arrays):
    return pl.pallas_call(
        _stats_kernel,
        out_shape=jax.ShapeDtypeStruct((8, INNER), jnp.float32),
    )(*arrays)


def _prep_kernel(nrows, tq_ref, tt_ref, wq_ref, wkv_ref, stq_ref, stt_ref,
                 g_ref, b_ref, q_ref, k_ref, v_ref):
    tq = tq_ref[0]
    tt = tt_ref[0]
    if stq_ref is not None:
        mean = (stq_ref[0:1, :] + stt_ref[0:1, :]) / nrows
        var = (stq_ref[1:2, :] + stt_ref[1:2, :]) / nrows - mean * mean
        inv = jax.lax.rsqrt(var + 1e-5) * g_ref[...]
        shift = b_ref[...] - mean * inv
        tq = tq * inv + shift
        tt = tt * inv + shift
    q_ref[0] = jnp.dot(tq, wq_ref[...], preferred_element_type=jnp.float32)
    kv = jnp.dot(tt, wkv_ref[...], preferred_element_type=jnp.float32)
    k_ref[0] = kv[:, :INNER]
    v_ref[0] = kv[:, INNER:]


def _prep_stage(tokens_q, tokens_t, wq, wkv, stats_q, stats_t, g, b):
    B, K, _ = tokens_q.shape
    T = tokens_t.shape[1]
    has_bn = stats_q is not None
    nrows = B * (K + T)
    in_specs = [
        pl.BlockSpec((1, K, INNER), lambda i: (i, 0, 0)),
        pl.BlockSpec((1, T, INNER), lambda i: (i, 0, 0)),
        pl.BlockSpec((INNER, INNER), lambda i: (0, 0)),
        pl.BlockSpec((INNER, 2 * INNER), lambda i: (0, 0)),
    ]
    args = [tokens_q, tokens_t, wq, wkv]
    if has_bn:
        in_specs += [
            pl.BlockSpec((8, INNER), lambda i: (0, 0)),
            pl.BlockSpec((8, INNER), lambda i: (0, 0)),
            pl.BlockSpec((1, INNER), lambda i: (0, 0)),
            pl.BlockSpec((1, INNER), lambda i: (0, 0)),
        ]
        args += [stats_q, stats_t, g.reshape(1, INNER), b.reshape(1, INNER)]
        body = functools.partial(_prep_kernel, nrows)
    else:
        body = (lambda tqr, ttr, wqr, wkvr, qr, kr, vr:
                _prep_kernel(nrows, tqr, ttr, wqr, wkvr, None, None, None,
                             None, qr, kr, vr))
    out_shapes = (
        jax.ShapeDtypeStruct((B, K, INNER), jnp.float32),
        jax.ShapeDtypeStruct((B, T, INNER), jnp.float32),
        jax.ShapeDtypeStruct((B, T, INNER), jnp.float32),
    )
    out_specs = (
        pl.BlockSpec((1, K, INNER), lambda i: (i, 0, 0)),
        pl.BlockSpec((1, T, INNER), lambda i: (i, 0, 0)),
        pl.BlockSpec((1, T, INNER), lambda i: (i, 0, 0)),
    )
    return pl.pallas_call(
        body, grid=(B,), in_specs=in_specs, out_specs=out_specs,
        out_shape=out_shapes,
    )(*args)


def _attn_kernel(k, scale, qx_ref, qy_ref, qz_ref, tx_ref, ty_ref, tz_ref,
                 q_ref, kx_ref, vx_ref, skip_ref, o_ref, st_ref,
                 dw_ref, nm_ref):
    qx = qx_ref[0]
    qy = qy_ref[0]
    qz = qz_ref[0]
    tx = tx_ref[0]
    ty = ty_ref[0]
    tz = tz_ref[0]
    T = tx.shape[1]
    n_q = qx.shape[0]
    sq = qx * qx + qy * qy + qz * qz
    st = tx * tx + ty * ty + tz * tz
    qmat = jnp.concatenate([qx, qy, qz], axis=1)
    tmat = jnp.concatenate([tx, ty, tz], axis=0)
    qt = jnp.dot(qmat, tmat, preferred_element_type=jnp.float32)
    d = (sq + st) - 2.0 * qt
    bits = jax.lax.bitcast_convert_type(d, jnp.int32)
    dw_ref[...] = jnp.where(bits < 0, bits ^ jnp.int32(0x7FFFFFFF), bits)

    def rbody(i, X):
        shift = jax.lax.shift_left(jnp.int32(1), jnp.int32(31) - i)
        c = X + shift
        cnt = jnp.sum((dw_ref[...] < c).astype(jnp.int32),
                      axis=1, keepdims=True)
        return jnp.where(cnt < k, c, X)

    X = jax.lax.fori_loop(
        0, 32, rbody, jnp.full((n_q, 1), jnp.int32(-2147483648)))
    skey = dw_ref[...]
    le = skey <= X
    cnt_le = jnp.sum(le.astype(jnp.int32), axis=1, keepdims=True)
    exact = jnp.all(cnt_le == k)

    @pl.when(exact)
    def _():
        nm_ref[...] = jnp.where(le, 0.0, -1e30)

    @pl.when(jnp.logical_not(exact))
    def _():
        less = skey < X
        cnt_less = jnp.sum(less.astype(jnp.int32), axis=1, keepdims=True)
        eq = (skey == X).astype(jnp.int32)
        s = eq
        sh = 1
        while sh < T:
            s = s + jnp.concatenate(
                [jnp.zeros((n_q, sh), jnp.int32), s[:, :T - sh]], axis=1)
            sh *= 2
        sel = less | ((eq == 1) & (s <= (k - cnt_less)))
        nm_ref[...] = jnp.where(sel, 0.0, -1e30)
    qk = jax.lax.dot_general(
        q_ref[0], kx_ref[0], (((1,), (1,)), ((), ())),
        preferred_element_type=jnp.float32)
    logits = qk * scale + nm_ref[...]
    mx = jnp.max(logits, axis=1, keepdims=True)
    e = jnp.exp(logits - mx)
    a = e / jnp.sum(e, axis=1, keepdims=True)
    av = jnp.dot(a, vx_ref[0], preferred_element_type=jnp.float32)
    o = av + skip_ref[0]
    o_ref[0] = o

    @pl.when((pl.program_id(0) == 0) & (pl.program_id(1) == 0))
    def _():
        st_ref[...] = jnp.zeros_like(st_ref)

    st_ref[0:1, :] = st_ref[0:1, :] + jnp.sum(o, axis=0, keepdims=True)
    st_ref[1:2, :] = st_ref[1:2, :] + jnp.sum(o * o, axis=0, keepdims=True)


def _attn_stage(qcoords, tcoords_lane, Q, Kx, Vx, skip, k):
    B, K, _ = Q.shape
    T = Kx.shape[1]
    scale = 1.0 / (float(INNER) ** 0.5)
    qspec = pl.BlockSpec((1, QT, 1), lambda b, t: (b, t, 0))
    tspec = pl.BlockSpec((1, 1, T), lambda b, t: (b, 0, 0))
    in_specs = [qspec, qspec, qspec, tspec, tspec, tspec,
                pl.BlockSpec((1, QT, INNER), lambda b, t: (b, t, 0)),
                pl.BlockSpec((1, T, INNER), lambda b, t: (b, 0, 0)),
                pl.BlockSpec((1, T, INNER), lambda b, t: (b, 0, 0)),
                pl.BlockSpec((1, QT, INNER), lambda b, t: (b, t, 0))]
    return pl.pallas_call(
        functools.partial(_attn_kernel, k, scale),
        grid=(B, K // QT),
        in_specs=in_specs,
        out_specs=(pl.BlockSpec((1, QT, INNER), lambda b, t: (b, t, 0)),
                   pl.BlockSpec((8, INNER), lambda b, t: (0, 0))),
        out_shape=(jax.ShapeDtypeStruct((B, K, INNER), jnp.float32),
                   jax.ShapeDtypeStruct((8, INNER), jnp.float32)),
        scratch_shapes=[pltpu.VMEM((QT, T), jnp.int32),
                        pltpu.VMEM((QT, T), jnp.float32)],
    )(*qcoords, *tcoords_lane, Q, Kx, Vx, skip)


def _tail_kernel(nrows, x_ref, st_ref, w1_ref, b1_ref, w2_ref, b2_ref,
                 g_ref, b_ref, o_ref, so_ref):
    x = x_ref[...]
    mean = st_ref[0:1, :] / nrows
    var = st_ref[1:2, :] / nrows - mean * mean
    inv = jax.lax.rsqrt(var + 1e-5) * g_ref[...]
    shift = b_ref[...] - mean * inv
    xn = x * inv + shift
    h = jnp.maximum(
        jnp.dot(xn, w1_ref[...], preferred_element_type=jnp.float32)
        + b1_ref[...], 0.0)
    y = (jnp.dot(h, w2_ref[...], preferred_element_type=jnp.float32)
         + b2_ref[...]) + x
    o_ref[...] = y

    @pl.when(pl.program_id(0) == 0)
    def _():
        so_ref[...] = jnp.zeros_like(so_ref)

    so_ref[0:1, :] = so_ref[0:1, :] + jnp.sum(y, axis=0, keepdims=True)
    so_ref[1:2, :] = so_ref[1:2, :] + jnp.sum(y * y, axis=0, keepdims=True)


def _tail_stage(x, stats, w1, b1, w2, b2, g, b):
    N = x.shape[0]
    R = 1024
    in_specs = [
        pl.BlockSpec((R, INNER), lambda i: (i, 0)),
        pl.BlockSpec((8, INNER), lambda i: (0, 0)),
        pl.BlockSpec((INNER, 2 * INNER), lambda i: (0, 0)),
        pl.BlockSpec((1, 2 * INNER), lambda i: (0, 0)),
        pl.BlockSpec((2 * INNER, INNER), lambda i: (0, 0)),
        pl.BlockSpec((1, INNER), lambda i: (0, 0)),
        pl.BlockSpec((1, INNER), lambda i: (0, 0)),
        pl.BlockSpec((1, INNER), lambda i: (0, 0)),
    ]
    return pl.pallas_call(
        functools.partial(_tail_kernel, N),
        grid=(N // R,),
        in_specs=in_specs,
        out_specs=(pl.BlockSpec((R, INNER), lambda i: (i, 0)),
                   pl.BlockSpec((8, INNER), lambda i: (0, 0))),
        out_shape=(jax.ShapeDtypeStruct((N, INNER), jnp.float32),
                   jax.ShapeDtypeStruct((8, INNER), jnp.float32)),
    )(x, stats, w1, b1.reshape(1, 2 * INNER), w2, b2.reshape(1, INNER),
      g.reshape(1, INNER), b.reshape(1, INNER))


def _final_kernel(x_ref, w_ref, b_ref, g_ref, beta_ref, o_ref):
    x = x_ref[...]
    e = jnp.dot(x, w_ref[...], preferred_element_type=jnp.float32) + b_ref[...]
    n = e.shape[0]
    m = jnp.sum(e, axis=0, keepdims=True) / n
    v = jnp.sum((e - m) ** 2, axis=0, keepdims=True) / n
    e = (e - m) / jnp.sqrt(v + 1e-5) * g_ref[...] + beta_ref[...]
    e = jnp.maximum(e, 0.0)
    B = o_ref.shape[0]
    e = e.reshape(B, n // B, e.shape[-1])
    o_ref[...] = jnp.max(e, axis=1)


def _final_stage(tokens_t, params):
    B, Tt, C = tokens_t.shape
    x = tokens_t.reshape(B * Tt, C)
    return pl.pallas_call(
        _final_kernel,
        out_shape=jax.ShapeDtypeStruct((B, EMB), jnp.float32),
    )(x, params['emb_w'], params['emb_b'].reshape(1, EMB),
      params['emb_g'].reshape(1, EMB), params['emb_beta'].reshape(1, EMB))


def kernel(tokens, centers, lrfs, params):
    B, T, C = tokens.shape
    tokens_t = tokens
    ccoords = tuple(
        centers[..., c].reshape(B, T // 128, 128) for c in range(3))
    T_t = T
    for i in range(NUM_BLOCKS):
        Tq = NUM_CENTER_PTS[i]
        k = NUM_NNS[i]
        rep_idx, qcoords = _fps_stage(ccoords, Tq)
        flat_idx = (rep_idx
                    + jnp.arange(B, dtype=jnp.int32)[:, None] * T_t
                    ).reshape(B * Tq)
        tokens_q = _sc_gather_rows(
            tokens_t.reshape(B * T_t, C), flat_idx).reshape(B, Tq, C)
        if i > 0:
            stats_q = _stats_stage([tokens_q.reshape(B * Tq, C)])
            g1, b1 = params['bn1_g%d' % i], params['bn1_b%d' % i]
        else:
            stats_q, prev_sums, g1, b1 = None, None, None, None
        Q, Kx, Vx = _prep_stage(tokens_q, tokens_t,
                                params['Wq%d' % i], params['Wkv%d' % i],
                                stats_q, prev_sums, g1, b1)
        tcoords_lane = tuple(cc.reshape(B, 1, T_t) for cc in ccoords)
        attn, stats2 = _attn_stage(
            qcoords, tcoords_lane, Q, Kx, Vx, tokens_q, k)
        x = attn.reshape(B * Tq, C)
        y, prev_sums = _tail_stage(
            x, stats2,
            params['mlp_w1_%d' % i], params['mlp_b1_%d' % i],
            params['mlp_w2_%d' % i], params['mlp_b2_%d' % i],
            params['bn2_g%d' % i], params['bn2_b%d' % i])
        tokens_t = y.reshape(B, Tq, C)
        ccoords = tuple(qc.reshape(B, Tq // 128, 128) for qc in qcoords)
        T_t = Tq
    return _final_stage(tokens_t, params)
